# baseline (speedup 1.0000x reference)
import functools
import math

import jax
import jax.numpy as jnp
from jax.experimental import pallas as pl
from jax.experimental.pallas import tpu as pltpu
from jax.experimental.pallas import tpu_sc as plsc

_B, _S, _D = 1, 2048, 2048
_H, _KVH, _HD = 16, 4, 128
_E, _K, _F = 8, 2, 768
_EPS = 1e-6
_THETA = 10000.0
_T = _B * _S
_A = _T * _K
_BT = 256
_NBLK = _A // _BT + _E
_APAD = _NBLK * _BT
_BQ = 512
_BSA = 256
_SCALE = 1.0 / math.sqrt(_HD)
_F32 = jnp.float32
_BF16 = jnp.bfloat16
_BSQ = 512


def _rms(x, w):
    return x * jax.lax.rsqrt(jnp.mean(x * x, axis=-1, keepdims=True) + _EPS) * w


def _qkv_body(x_ref, ln1_ref, wq_ref, wk_ref, wv_ref, q_ref, k_ref, v_ref):
    x = x_ref[...]
    h = _rms(x, ln1_ref[...]).astype(_BF16)
    q_ref[...] = jnp.dot(h, wq_ref[...].astype(_BF16),
                         preferred_element_type=_F32)
    k_ref[...] = jnp.dot(h, wk_ref[...].astype(_BF16),
                         preferred_element_type=_F32)
    v_ref[...] = jnp.dot(h, wv_ref[...].astype(_BF16),
                         preferred_element_type=_F32)


def _qkv_call(x, ln1_w, wq, wk, wv):
    grid = (_S // _BSQ,)
    return pl.pallas_call(
        _qkv_body,
        grid=grid,
        in_specs=[
            pl.BlockSpec((_BSQ, _D), lambda i: (i, 0)),
            pl.BlockSpec((1, _D), lambda i: (0, 0)),
            pl.BlockSpec((_D, _H * _HD), lambda i: (0, 0)),
            pl.BlockSpec((_D, _KVH * _HD), lambda i: (0, 0)),
            pl.BlockSpec((_D, _KVH * _HD), lambda i: (0, 0)),
        ],
        out_specs=[
            pl.BlockSpec((_BSQ, _H * _HD), lambda i: (i, 0)),
            pl.BlockSpec((_BSQ, _KVH * _HD), lambda i: (i, 0)),
            pl.BlockSpec((_BSQ, _KVH * _HD), lambda i: (i, 0)),
        ],
        out_shape=[
            jax.ShapeDtypeStruct((_S, _H * _HD), _F32),
            jax.ShapeDtypeStruct((_S, _KVH * _HD), _F32),
            jax.ShapeDtypeStruct((_S, _KVH * _HD), _F32),
        ],
        compiler_params=pltpu.CompilerParams(
            vmem_limit_bytes=100 * 1024 * 1024),
    )(x, ln1_w.reshape(1, _D), wq, wk, wv)


def _prep_body(x_ref, nw_ref, o_ref):
    half = _HD // 2
    x = _rms(x_ref[...], nw_ref[...])
    pos = jax.lax.broadcasted_iota(jnp.int32, (_S, 1), 0).astype(_F32)
    inv = jnp.exp(jax.lax.broadcasted_iota(jnp.int32, (1, half), 1)
                  .astype(_F32) * (-math.log(_THETA) / half))
    f = pos * inv
    cos = jnp.cos(f)
    sin = jnp.sin(f)
    x1 = x[:, :half]
    x2 = x[:, half:]
    o_ref[0] = jnp.concatenate([x1 * cos - x2 * sin, x2 * cos + x1 * sin],
                               axis=-1).astype(_BF16)


def _prep_call(x2d, nw, nh):
    return pl.pallas_call(
        _prep_body,
        grid=(nh,),
        in_specs=[
            pl.BlockSpec((_S, _HD), lambda h: (0, h)),
            pl.BlockSpec((1, _HD), lambda h: (0, 0)),
        ],
        out_specs=pl.BlockSpec((1, _S, _HD), lambda h: (h, 0, 0)),
        out_shape=jax.ShapeDtypeStruct((nh, _S, _HD), _BF16),
        compiler_params=pltpu.CompilerParams(
            vmem_limit_bytes=100 * 1024 * 1024),
    )(x2d, nw.reshape(1, _HD))


def _attn_body(q_ref, k_ref, v_ref, o_ref):
    qb = pl.program_id(1)
    q = q_ref[0]

    def step(j, carry, masked):
        m, l, acc = carry
        k = k_ref[0, pl.ds(j * _BQ, _BQ), :]
        s = jax.lax.dot_general(q, k, (((1,), (1,)), ((), ())),
                                preferred_element_type=_F32) * _SCALE
        if masked:
            row = jax.lax.broadcasted_iota(jnp.int32, (_BQ, _BQ), 0)
            col = jax.lax.broadcasted_iota(jnp.int32, (_BQ, _BQ), 1)
            s = jnp.where(row >= col, s, -1e30)
        mj = jnp.max(s, axis=-1, keepdims=True)
        mn = jnp.maximum(m, mj)
        p = jnp.exp(s - mn)
        c = jnp.exp(m - mn)
        v = v_ref[0, pl.ds(j * _BQ, _BQ), :]
        acc = acc * c + jnp.dot(p.astype(_BF16), v,
                                preferred_element_type=_F32)
        l = l * c + jnp.sum(p, axis=-1, keepdims=True)
        return mn, l, acc

    init = (jnp.full((_BQ, 1), -1e30, _F32),
            jnp.zeros((_BQ, 1), _F32),
            jnp.zeros((_BQ, _HD), _F32))
    m, l, acc = jax.lax.fori_loop(
        0, qb, lambda j, cr: step(j, cr, False), init)
    m, l, acc = step(qb, (m, l, acc), True)
    o_ref[0] = (acc / l).astype(_BF16)


def _attn_call(q, k, v):
    rep = _H // _KVH
    grid = (_H, _S // _BQ)
    return pl.pallas_call(
        _attn_body,
        grid=grid,
        in_specs=[
            pl.BlockSpec((1, _BQ, _HD), lambda h, qb: (h, qb, 0)),
            pl.BlockSpec((1, _S, _HD), lambda h, qb: (h // rep, 0, 0)),
            pl.BlockSpec((1, _S, _HD), lambda h, qb: (h // rep, 0, 0)),
        ],
        out_specs=pl.BlockSpec((1, _BQ, _HD), lambda h, qb: (h, qb, 0)),
        out_shape=jax.ShapeDtypeStruct((_H, _S, _HD), _BF16),
        compiler_params=pltpu.CompilerParams(
            vmem_limit_bytes=100 * 1024 * 1024),
    )(q, k, v)


def _proj_body(o_ref, wo_ref, res_ref, out_ref):
    out_ref[...] = res_ref[...] + jnp.dot(o_ref[...],
                                          wo_ref[...].astype(_BF16),
                                          preferred_element_type=_F32)


def _proj_call(o, wo, res):
    grid = (_S // _BSQ,)
    return pl.pallas_call(
        _proj_body,
        grid=grid,
        in_specs=[
            pl.BlockSpec((_BSQ, _H * _HD), lambda i: (i, 0)),
            pl.BlockSpec((_H * _HD, _D), lambda i: (0, 0)),
            pl.BlockSpec((_BSQ, _D), lambda i: (i, 0)),
        ],
        out_specs=pl.BlockSpec((_BSQ, _D), lambda i: (i, 0)),
        out_shape=jax.ShapeDtypeStruct((_T, _D), _F32),
        compiler_params=pltpu.CompilerParams(
            vmem_limit_bytes=100 * 1024 * 1024),
    )(o, wo, res)


def _router_body(x_ref, ln2_ref, rw_ref, h_ref, w1_ref, w2_ref, i1_ref,
                 i2_ref):
    x = x_ref[...]
    h = _rms(x, ln2_ref[...])
    h_ref[...] = h.astype(_BF16)
    logits = jnp.dot(h, rw_ref[...], preferred_element_type=_F32)
    m = jnp.max(logits, axis=-1, keepdims=True)
    p = jnp.exp(logits - m)
    p = p / jnp.sum(p, axis=-1, keepdims=True)
    ids = jax.lax.broadcasted_iota(jnp.int32, (_BSA, _E), 1)
    m1 = jnp.max(p, axis=-1, keepdims=True)
    i1 = jnp.min(jnp.where(p == m1, ids, _E), axis=-1, keepdims=True)
    pm = jnp.where(ids == i1, -1.0, p)
    m2 = jnp.max(pm, axis=-1, keepdims=True)
    i2 = jnp.min(jnp.where(pm == m2, ids, _E), axis=-1, keepdims=True)
    sw = m1 + m2
    w1_ref[...] = m1 / sw
    w2_ref[...] = m2 / sw
    i1_ref[...] = i1
    i2_ref[...] = i2


def _router_call(x, ln2_w, rw):
    grid = (_S // _BSA,)
    return pl.pallas_call(
        _router_body,
        grid=grid,
        in_specs=[
            pl.BlockSpec((_BSA, _D), lambda i: (i, 0)),
            pl.BlockSpec((1, _D), lambda i: (0, 0)),
            pl.BlockSpec((_D, _E), lambda i: (0, 0)),
        ],
        out_specs=[
            pl.BlockSpec((_BSA, _D), lambda i: (i, 0)),
            pl.BlockSpec((_BSA, 1), lambda i: (i, 0)),
            pl.BlockSpec((_BSA, 1), lambda i: (i, 0)),
            pl.BlockSpec((_BSA, 1), lambda i: (i, 0)),
            pl.BlockSpec((_BSA, 1), lambda i: (i, 0)),
        ],
        out_shape=[
            jax.ShapeDtypeStruct((_T, _D), _BF16),
            jax.ShapeDtypeStruct((_T, 1), _F32),
            jax.ShapeDtypeStruct((_T, 1), _F32),
            jax.ShapeDtypeStruct((_T, 1), jnp.int32),
            jax.ShapeDtypeStruct((_T, 1), jnp.int32),
        ],
        compiler_params=pltpu.CompilerParams(
            vmem_limit_bytes=100 * 1024 * 1024),
    )(x, ln2_w.reshape(1, _D), rw)


def _moe_body(meta_ref, x_ref, wg_ref, wu_ref, wd_ref, y_ref,
              wg16_ref, wu16_ref, wd16_ref):
    b = pl.program_id(0)
    live = meta_ref[1, b] > 0
    changed = jnp.logical_or(
        b == 0, meta_ref[0, b] != meta_ref[0, jnp.maximum(b - 1, 0)])

    @pl.when(jnp.logical_and(live, changed))
    def _():
        wg16_ref[...] = wg_ref[0].astype(_BF16)
        wu16_ref[...] = wu_ref[0].astype(_BF16)
        wd16_ref[...] = wd_ref[0].astype(_BF16)

    @pl.when(live)
    def _():
        x = x_ref[...]
        g = jnp.dot(x, wg16_ref[...], preferred_element_type=_F32)
        u = jnp.dot(x, wu16_ref[...], preferred_element_type=_F32)
        a = (g * jax.lax.logistic(g) * u).astype(_BF16)
        y_ref[...] = jnp.dot(a, wd16_ref[...],
                             preferred_element_type=_F32).astype(_BF16)

    @pl.when(jnp.logical_not(live))
    def _():
        y_ref[...] = jnp.zeros_like(y_ref)


def _moe_call(meta, x_sorted, wg, wu, wd):
    grid_spec = pltpu.PrefetchScalarGridSpec(
        num_scalar_prefetch=1,
        grid=(_NBLK,),
        in_specs=[
            pl.BlockSpec((_BT, _D), lambda b, m: (b, 0)),
            pl.BlockSpec((1, _D, _F), lambda b, m: (m[0, b], 0, 0)),
            pl.BlockSpec((1, _D, _F), lambda b, m: (m[0, b], 0, 0)),
            pl.BlockSpec((1, _F, _D), lambda b, m: (m[0, b], 0, 0)),
        ],
        out_specs=pl.BlockSpec((_BT, _D), lambda b, m: (b, 0)),
        scratch_shapes=[
            pltpu.VMEM((_D, _F), _BF16),
            pltpu.VMEM((_D, _F), _BF16),
            pltpu.VMEM((_F, _D), _BF16),
        ],
    )
    return pl.pallas_call(
        _moe_body,
        grid_spec=grid_spec,
        out_shape=jax.ShapeDtypeStruct((_APAD, _D), _BF16),
        compiler_params=pltpu.CompilerParams(
            dimension_semantics=("arbitrary",),
            vmem_limit_bytes=110 * 1024 * 1024),
    )(meta, x_sorted, wg, wu, wd)


_NC, _NS = 2, 16
_NW = _NC * _NS
_DW = _D // 2


def _sc_gather(idx2d, table_i32, n_rows, chunk):
    per_w = n_rows // _NW
    n_ch = per_w // chunk
    mesh = plsc.VectorSubcoreMesh(core_axis_name="c", subcore_axis_name="s")

    @functools.partial(
        pl.kernel, mesh=mesh,
        out_type=jax.ShapeDtypeStruct((n_rows, _DW), jnp.int32),
        scratch_types=[
            pltpu.VMEM((n_ch, chunk), jnp.int32),
            pltpu.VMEM((2, chunk, _DW), jnp.int32),
            pltpu.SemaphoreType.DMA((2,)),
            pltpu.SemaphoreType.DMA((2,)),
        ],
    )
    def gath(idx_hbm, tab_hbm, out_hbm, idx_v, buf_v, gsem, wsem):
        wid = jax.lax.axis_index("s") * _NC + jax.lax.axis_index("c")
        base = wid * per_w
        pltpu.sync_copy(idx_hbm.at[pl.ds(wid * n_ch, n_ch)], idx_v)
        gh = {}
        wh = {}

        def fire(c):
            gh[c] = pltpu.async_copy(tab_hbm.at[idx_v.at[c]],
                                     buf_v.at[c % 2], gsem.at[c % 2])

        fire(0)
        for c in range(n_ch):
            if c + 1 < n_ch:
                if c - 1 >= 0:
                    wh[c - 1].wait()
                fire(c + 1)
            gh[c].wait()
            wh[c] = pltpu.async_copy(
                buf_v.at[c % 2],
                out_hbm.at[pl.ds(base + c * chunk, chunk)],
                wsem.at[c % 2])
        for c in range(max(0, n_ch - 2), n_ch):
            wh[c].wait()

    return gath(idx2d, table_i32)


def _as_words(x16):
    n = x16.shape[0]
    return jax.lax.bitcast_convert_type(x16.reshape(n, _DW, 2), jnp.int32)


def _as_bf16(xi):
    n = xi.shape[0]
    return jax.lax.bitcast_convert_type(xi, _BF16).reshape(n, _D)


def _comb_body(x2_ref, y0_ref, y1_ref, w1_ref, w2_ref, out_ref):
    out_ref[...] = (x2_ref[...]
                    + w1_ref[...] * y0_ref[...].astype(_F32)
                    + w2_ref[...] * y1_ref[...].astype(_F32))


def _comb_call(x2, y0, y1, w1, w2):
    grid = (_S // _BSQ,)
    return pl.pallas_call(
        _comb_body,
        grid=grid,
        in_specs=[
            pl.BlockSpec((_BSQ, _D), lambda i: (i, 0)),
            pl.BlockSpec((_BSQ, _D), lambda i: (i, 0)),
            pl.BlockSpec((_BSQ, _D), lambda i: (i, 0)),
            pl.BlockSpec((_BSQ, 1), lambda i: (i, 0)),
            pl.BlockSpec((_BSQ, 1), lambda i: (i, 0)),
        ],
        out_specs=pl.BlockSpec((_BSQ, _D), lambda i: (i, 0)),
        out_shape=jax.ShapeDtypeStruct((_T, _D), _F32),
        compiler_params=pltpu.CompilerParams(
            vmem_limit_bytes=100 * 1024 * 1024),
    )(x2, y0, y1, w1, w2)


def _dispatch(i1, i2):
    topi = jnp.concatenate([i1, i2], axis=1)
    flat_e = topi.reshape(_A)
    onehot = (flat_e[:, None] == jnp.arange(_E)[None, :]).astype(jnp.int32)
    rank = jnp.take_along_axis(jnp.cumsum(onehot, axis=0) - onehot,
                               flat_e[:, None], axis=1)[:, 0]
    counts = jnp.sum(onehot, axis=0)
    nblk_e = (counts + _BT - 1) // _BT
    end_blk = jnp.cumsum(nblk_e)
    used = end_blk[-1]
    start_row = jnp.concatenate(
        [jnp.zeros((1,), jnp.int32), jnp.cumsum(nblk_e * _BT)[:-1]])
    pos = start_row[flat_e] + rank
    tok = jnp.arange(_A, dtype=jnp.int32) // _K
    tok_sorted = jnp.zeros((_APAD,), jnp.int32).at[pos].set(tok)
    bidx = jnp.arange(_NBLK, dtype=jnp.int32)
    eob = jnp.sum((bidx[:, None] >= end_blk[None, :]).astype(jnp.int32),
                  axis=1)
    is_real = (bidx < used).astype(jnp.int32)
    eob_last = jnp.sum((end_blk <= used - 1).astype(jnp.int32))
    eob = jnp.where(is_real > 0, eob, eob_last)
    meta = jnp.stack([eob, is_real]).astype(jnp.int32)
    return tok_sorted, meta, pos.reshape(_T, _K).astype(jnp.int32)


def kernel(hidden_states, ln1_w, Wq, Wk, Wv, q_norm_w, k_norm_w, Wo, ln2_w,
           router_W, W_gate, W_up, W_down):
    x = hidden_states.reshape(_T, _D)
    q2, k2, v2 = _qkv_call(x, ln1_w, Wq, Wk, Wv)
    q = _prep_call(q2, q_norm_w, _H)
    k = _prep_call(k2, k_norm_w, _KVH)
    v = v2.reshape(_S, _KVH, _HD).transpose(1, 0, 2).astype(_BF16)
    o = _attn_call(q, k, v)
    x2 = _proj_call(o.transpose(1, 0, 2).reshape(_S, _H * _HD), Wo, x)
    h2, w1, w2, i1, i2 = _router_call(x2, ln2_w, router_W)
    tok_sorted, meta, posr = _dispatch(i1, i2)
    h2w = _as_words(h2)
    xs_w = _sc_gather(tok_sorted.reshape(_APAD // 24, 24), h2w, _APAD, 24)
    x_sorted = _as_bf16(xs_w)
    y_sorted = _moe_call(meta, x_sorted, W_gate, W_up, W_down)
    yw = _as_words(y_sorted)
    y0 = _as_bf16(_sc_gather(posr[:, 0].reshape(_T // 16, 16), yw, _T, 16))
    y1 = _as_bf16(_sc_gather(posr[:, 1].reshape(_T // 16, 16), yw, _T, 16))
    out = _comb_call(x2, y0, y1, w1, w2)
    return out.reshape(_B, _S, _D)


# speedup vs baseline: 1.9507x; 1.9507x over previous
import functools
import math

import jax
import jax.numpy as jnp
from jax.experimental import pallas as pl
from jax.experimental.pallas import tpu as pltpu
from jax.experimental.pallas import tpu_sc as plsc

_B, _S, _D = 1, 2048, 2048
_H, _KVH, _HD = 16, 4, 128
_E, _K, _F = 8, 2, 768
_EPS = 1e-6
_THETA = 10000.0
_T = _B * _S
_A = _T * _K
_BT = 256
_NBLK = _A // _BT + _E
_APAD = _NBLK * _BT
_BQ = 512
_BSA = 256
_SCALE = 1.0 / math.sqrt(_HD)
_F32 = jnp.float32
_BF16 = jnp.bfloat16
_BSQ = 512


def _rms(x, w):
    return x * jax.lax.rsqrt(jnp.mean(x * x, axis=-1, keepdims=True) + _EPS) * w


def _qkv_body(x_ref, ln1_ref, wq_ref, wk_ref, wv_ref, q_ref, k_ref, v_ref):
    x = x_ref[...]
    h = _rms(x, ln1_ref[...]).astype(_BF16)
    q_ref[...] = jnp.dot(h, wq_ref[...].astype(_BF16),
                         preferred_element_type=_F32)
    k_ref[...] = jnp.dot(h, wk_ref[...].astype(_BF16),
                         preferred_element_type=_F32)
    v_ref[...] = jnp.dot(h, wv_ref[...].astype(_BF16),
                         preferred_element_type=_F32)


def _qkv_call(x, ln1_w, wq, wk, wv):
    grid = (_S // _BSQ,)
    return pl.pallas_call(
        _qkv_body,
        grid=grid,
        in_specs=[
            pl.BlockSpec((_BSQ, _D), lambda i: (i, 0)),
            pl.BlockSpec((1, _D), lambda i: (0, 0)),
            pl.BlockSpec((_D, _H * _HD), lambda i: (0, 0)),
            pl.BlockSpec((_D, _KVH * _HD), lambda i: (0, 0)),
            pl.BlockSpec((_D, _KVH * _HD), lambda i: (0, 0)),
        ],
        out_specs=[
            pl.BlockSpec((_BSQ, _H * _HD), lambda i: (i, 0)),
            pl.BlockSpec((_BSQ, _KVH * _HD), lambda i: (i, 0)),
            pl.BlockSpec((_BSQ, _KVH * _HD), lambda i: (i, 0)),
        ],
        out_shape=[
            jax.ShapeDtypeStruct((_S, _H * _HD), _F32),
            jax.ShapeDtypeStruct((_S, _KVH * _HD), _F32),
            jax.ShapeDtypeStruct((_S, _KVH * _HD), _F32),
        ],
        compiler_params=pltpu.CompilerParams(
            vmem_limit_bytes=100 * 1024 * 1024),
    )(x, ln1_w.reshape(1, _D), wq, wk, wv)


def _prep_body(x_ref, nw_ref, o_ref):
    half = _HD // 2
    x = _rms(x_ref[...], nw_ref[...])
    pos = jax.lax.broadcasted_iota(jnp.int32, (_S, 1), 0).astype(_F32)
    inv = jnp.exp(jax.lax.broadcasted_iota(jnp.int32, (1, half), 1)
                  .astype(_F32) * (-math.log(_THETA) / half))
    f = pos * inv
    cos = jnp.cos(f)
    sin = jnp.sin(f)
    x1 = x[:, :half]
    x2 = x[:, half:]
    o_ref[0] = jnp.concatenate([x1 * cos - x2 * sin, x2 * cos + x1 * sin],
                               axis=-1).astype(_BF16)


def _prep_call(x2d, nw, nh):
    return pl.pallas_call(
        _prep_body,
        grid=(nh,),
        in_specs=[
            pl.BlockSpec((_S, _HD), lambda h: (0, h)),
            pl.BlockSpec((1, _HD), lambda h: (0, 0)),
        ],
        out_specs=pl.BlockSpec((1, _S, _HD), lambda h: (h, 0, 0)),
        out_shape=jax.ShapeDtypeStruct((nh, _S, _HD), _BF16),
        compiler_params=pltpu.CompilerParams(
            vmem_limit_bytes=100 * 1024 * 1024),
    )(x2d, nw.reshape(1, _HD))


def _attn_body(q_ref, k_ref, v_ref, o_ref):
    qb = pl.program_id(1)
    q = q_ref[0]

    def step(j, carry, masked):
        m, l, acc = carry
        k = k_ref[0, pl.ds(j * _BQ, _BQ), :]
        s = jax.lax.dot_general(q, k, (((1,), (1,)), ((), ())),
                                preferred_element_type=_F32) * _SCALE
        if masked:
            row = jax.lax.broadcasted_iota(jnp.int32, (_BQ, _BQ), 0)
            col = jax.lax.broadcasted_iota(jnp.int32, (_BQ, _BQ), 1)
            s = jnp.where(row >= col, s, -1e30)
        mj = jnp.max(s, axis=-1, keepdims=True)
        mn = jnp.maximum(m, mj)
        p = jnp.exp(s - mn)
        c = jnp.exp(m - mn)
        v = v_ref[0, pl.ds(j * _BQ, _BQ), :]
        acc = acc * c + jnp.dot(p.astype(_BF16), v,
                                preferred_element_type=_F32)
        l = l * c + jnp.sum(p, axis=-1, keepdims=True)
        return mn, l, acc

    init = (jnp.full((_BQ, 1), -1e30, _F32),
            jnp.zeros((_BQ, 1), _F32),
            jnp.zeros((_BQ, _HD), _F32))
    m, l, acc = jax.lax.fori_loop(
        0, qb, lambda j, cr: step(j, cr, False), init)
    m, l, acc = step(qb, (m, l, acc), True)
    o_ref[0] = (acc / l).astype(_BF16)


def _attn_call(q, k, v):
    rep = _H // _KVH
    grid = (_H, _S // _BQ)
    return pl.pallas_call(
        _attn_body,
        grid=grid,
        in_specs=[
            pl.BlockSpec((1, _BQ, _HD), lambda h, qb: (h, qb, 0)),
            pl.BlockSpec((1, _S, _HD), lambda h, qb: (h // rep, 0, 0)),
            pl.BlockSpec((1, _S, _HD), lambda h, qb: (h // rep, 0, 0)),
        ],
        out_specs=pl.BlockSpec((1, _BQ, _HD), lambda h, qb: (h, qb, 0)),
        out_shape=jax.ShapeDtypeStruct((_H, _S, _HD), _BF16),
        compiler_params=pltpu.CompilerParams(
            vmem_limit_bytes=100 * 1024 * 1024),
    )(q, k, v)


def _proj_body(o_ref, wo_ref, res_ref, out_ref):
    out_ref[...] = res_ref[...] + jnp.dot(o_ref[...],
                                          wo_ref[...].astype(_BF16),
                                          preferred_element_type=_F32)


def _proj_call(o, wo, res):
    grid = (_S // _BSQ,)
    return pl.pallas_call(
        _proj_body,
        grid=grid,
        in_specs=[
            pl.BlockSpec((_BSQ, _H * _HD), lambda i: (i, 0)),
            pl.BlockSpec((_H * _HD, _D), lambda i: (0, 0)),
            pl.BlockSpec((_BSQ, _D), lambda i: (i, 0)),
        ],
        out_specs=pl.BlockSpec((_BSQ, _D), lambda i: (i, 0)),
        out_shape=jax.ShapeDtypeStruct((_T, _D), _F32),
        compiler_params=pltpu.CompilerParams(
            vmem_limit_bytes=100 * 1024 * 1024),
    )(o, wo, res)


def _router_body(x_ref, ln2_ref, rw_ref, h_ref, w1_ref, w2_ref, i1_ref,
                 i2_ref):
    x = x_ref[...]
    h = _rms(x, ln2_ref[...])
    h_ref[...] = h
    logits = jnp.dot(h, rw_ref[...], preferred_element_type=_F32)
    m = jnp.max(logits, axis=-1, keepdims=True)
    p = jnp.exp(logits - m)
    p = p / jnp.sum(p, axis=-1, keepdims=True)
    ids = jax.lax.broadcasted_iota(jnp.int32, (_BSA, _E), 1)
    m1 = jnp.max(p, axis=-1, keepdims=True)
    i1 = jnp.min(jnp.where(p == m1, ids, _E), axis=-1, keepdims=True)
    pm = jnp.where(ids == i1, -1.0, p)
    m2 = jnp.max(pm, axis=-1, keepdims=True)
    i2 = jnp.min(jnp.where(pm == m2, ids, _E), axis=-1, keepdims=True)
    sw = m1 + m2
    w1_ref[...] = m1 / sw
    w2_ref[...] = m2 / sw
    i1_ref[...] = i1
    i2_ref[...] = i2


def _router_call(x, ln2_w, rw):
    grid = (_S // _BSA,)
    return pl.pallas_call(
        _router_body,
        grid=grid,
        in_specs=[
            pl.BlockSpec((_BSA, _D), lambda i: (i, 0)),
            pl.BlockSpec((1, _D), lambda i: (0, 0)),
            pl.BlockSpec((_D, _E), lambda i: (0, 0)),
        ],
        out_specs=[
            pl.BlockSpec((_BSA, _D), lambda i: (i, 0)),
            pl.BlockSpec((_BSA, 1), lambda i: (i, 0)),
            pl.BlockSpec((_BSA, 1), lambda i: (i, 0)),
            pl.BlockSpec((_BSA, 1), lambda i: (i, 0)),
            pl.BlockSpec((_BSA, 1), lambda i: (i, 0)),
        ],
        out_shape=[
            jax.ShapeDtypeStruct((_T, _D), _F32),
            jax.ShapeDtypeStruct((_T, 1), _F32),
            jax.ShapeDtypeStruct((_T, 1), _F32),
            jax.ShapeDtypeStruct((_T, 1), jnp.int32),
            jax.ShapeDtypeStruct((_T, 1), jnp.int32),
        ],
        compiler_params=pltpu.CompilerParams(
            vmem_limit_bytes=100 * 1024 * 1024),
    )(x, ln2_w.reshape(1, _D), rw)


def _moe_body(meta_ref, x_ref, wg_ref, wu_ref, wd_ref, y_ref,
              wg16_ref, wu16_ref, wd16_ref):
    b = pl.program_id(0)
    live = meta_ref[1, b] > 0
    changed = jnp.logical_or(
        b == 0, meta_ref[0, b] != meta_ref[0, jnp.maximum(b - 1, 0)])

    @pl.when(jnp.logical_and(live, changed))
    def _():
        wg16_ref[...] = wg_ref[0].astype(_BF16)
        wu16_ref[...] = wu_ref[0].astype(_BF16)
        wd16_ref[...] = wd_ref[0].astype(_BF16)

    @pl.when(live)
    def _():
        x = x_ref[...].astype(_BF16)
        g = jnp.dot(x, wg16_ref[...], preferred_element_type=_F32)
        u = jnp.dot(x, wu16_ref[...], preferred_element_type=_F32)
        a = (g * jax.lax.logistic(g) * u).astype(_BF16)
        y_ref[...] = jnp.dot(a, wd16_ref[...], preferred_element_type=_F32)

    @pl.when(jnp.logical_not(live))
    def _():
        y_ref[...] = jnp.zeros_like(y_ref)


def _moe_call(meta, x_sorted, wg, wu, wd):
    grid_spec = pltpu.PrefetchScalarGridSpec(
        num_scalar_prefetch=1,
        grid=(_NBLK,),
        in_specs=[
            pl.BlockSpec((_BT, _D), lambda b, m: (b, 0)),
            pl.BlockSpec((1, _D, _F), lambda b, m: (m[0, b], 0, 0)),
            pl.BlockSpec((1, _D, _F), lambda b, m: (m[0, b], 0, 0)),
            pl.BlockSpec((1, _F, _D), lambda b, m: (m[0, b], 0, 0)),
        ],
        out_specs=pl.BlockSpec((_BT, _D), lambda b, m: (b, 0)),
        scratch_shapes=[
            pltpu.VMEM((_D, _F), _BF16),
            pltpu.VMEM((_D, _F), _BF16),
            pltpu.VMEM((_F, _D), _BF16),
        ],
    )
    return pl.pallas_call(
        _moe_body,
        grid_spec=grid_spec,
        out_shape=jax.ShapeDtypeStruct((_APAD, _D), _F32),
        compiler_params=pltpu.CompilerParams(
            dimension_semantics=("arbitrary",),
            vmem_limit_bytes=110 * 1024 * 1024),
    )(meta, x_sorted, wg, wu, wd)


_NC, _NS = 2, 16
_NW = _NC * _NS
_DW = _D


def _sc_gather(idx2d, table, n_rows, chunk):
    per_w = n_rows // _NW
    n_ch = per_w // chunk
    mesh = plsc.VectorSubcoreMesh(core_axis_name="c", subcore_axis_name="s")

    @functools.partial(
        pl.kernel, mesh=mesh,
        out_type=jax.ShapeDtypeStruct((n_rows, _DW), _F32),
        scratch_types=[
            pltpu.VMEM((n_ch, chunk), jnp.int32),
            pltpu.VMEM((2, chunk, _DW), _F32),
            pltpu.SemaphoreType.DMA((2,)),
            pltpu.SemaphoreType.DMA((2,)),
        ],
    )
    def gath(idx_hbm, tab_hbm, out_hbm, idx_v, buf_v, gsem, wsem):
        wid = jax.lax.axis_index("s") * _NC + jax.lax.axis_index("c")
        base = wid * per_w
        pltpu.sync_copy(idx_hbm.at[pl.ds(wid * n_ch, n_ch)], idx_v)
        gh = {}
        wh = {}

        def fire(c):
            gh[c] = pltpu.async_copy(tab_hbm.at[idx_v.at[c]],
                                     buf_v.at[c % 2], gsem.at[c % 2])

        fire(0)
        for c in range(n_ch):
            if c + 1 < n_ch:
                if c - 1 >= 0:
                    wh[c - 1].wait()
                fire(c + 1)
            gh[c].wait()
            wh[c] = pltpu.async_copy(
                buf_v.at[c % 2],
                out_hbm.at[pl.ds(base + c * chunk, chunk)],
                wsem.at[c % 2])
        for c in range(max(0, n_ch - 2), n_ch):
            wh[c].wait()

    return gath(idx2d, table)


def _comb_body(x2_ref, y0_ref, y1_ref, w1_ref, w2_ref, out_ref):
    out_ref[...] = (x2_ref[...] + w1_ref[...] * y0_ref[...]
                    + w2_ref[...] * y1_ref[...])


def _comb_call(x2, y0, y1, w1, w2):
    grid = (_S // _BSQ,)
    return pl.pallas_call(
        _comb_body,
        grid=grid,
        in_specs=[
            pl.BlockSpec((_BSQ, _D), lambda i: (i, 0)),
            pl.BlockSpec((_BSQ, _D), lambda i: (i, 0)),
            pl.BlockSpec((_BSQ, _D), lambda i: (i, 0)),
            pl.BlockSpec((_BSQ, 1), lambda i: (i, 0)),
            pl.BlockSpec((_BSQ, 1), lambda i: (i, 0)),
        ],
        out_specs=pl.BlockSpec((_BSQ, _D), lambda i: (i, 0)),
        out_shape=jax.ShapeDtypeStruct((_T, _D), _F32),
        compiler_params=pltpu.CompilerParams(
            vmem_limit_bytes=100 * 1024 * 1024),
    )(x2, y0, y1, w1, w2)


def _dispatch(i1, i2):
    topi = jnp.concatenate([i1, i2], axis=1)
    flat_e = topi.reshape(_A)
    onehot = (flat_e[:, None] == jnp.arange(_E)[None, :]).astype(jnp.int32)
    rank = jnp.take_along_axis(jnp.cumsum(onehot, axis=0) - onehot,
                               flat_e[:, None], axis=1)[:, 0]
    counts = jnp.sum(onehot, axis=0)
    nblk_e = (counts + _BT - 1) // _BT
    end_blk = jnp.cumsum(nblk_e)
    used = end_blk[-1]
    start_row = jnp.concatenate(
        [jnp.zeros((1,), jnp.int32), jnp.cumsum(nblk_e * _BT)[:-1]])
    pos = start_row[flat_e] + rank
    tok = jnp.arange(_A, dtype=jnp.int32) // _K
    tok_sorted = jnp.zeros((_APAD,), jnp.int32).at[pos].set(tok)
    bidx = jnp.arange(_NBLK, dtype=jnp.int32)
    eob = jnp.sum((bidx[:, None] >= end_blk[None, :]).astype(jnp.int32),
                  axis=1)
    is_real = (bidx < used).astype(jnp.int32)
    eob_last = jnp.sum((end_blk <= used - 1).astype(jnp.int32))
    eob = jnp.where(is_real > 0, eob, eob_last)
    meta = jnp.stack([eob, is_real]).astype(jnp.int32)
    return tok_sorted, meta, pos.reshape(_T, _K).astype(jnp.int32)


def kernel(hidden_states, ln1_w, Wq, Wk, Wv, q_norm_w, k_norm_w, Wo, ln2_w,
           router_W, W_gate, W_up, W_down):
    x = hidden_states.reshape(_T, _D)
    q2, k2, v2 = _qkv_call(x, ln1_w, Wq, Wk, Wv)
    q = _prep_call(q2, q_norm_w, _H)
    k = _prep_call(k2, k_norm_w, _KVH)
    v = v2.reshape(_S, _KVH, _HD).transpose(1, 0, 2).astype(_BF16)
    o = _attn_call(q, k, v)
    x2 = _proj_call(o.transpose(1, 0, 2).reshape(_S, _H * _HD), Wo, x)
    h2, w1, w2, i1, i2 = _router_call(x2, ln2_w, router_W)
    tok_sorted, meta, posr = _dispatch(i1, i2)
    x_sorted = _sc_gather(tok_sorted.reshape(_APAD // 8, 8), h2, _APAD, 8)
    y_sorted = _moe_call(meta, x_sorted, W_gate, W_up, W_down)
    y0 = _sc_gather(posr[:, 0].reshape(_T // 8, 8), y_sorted, _T, 8)
    y1 = _sc_gather(posr[:, 1].reshape(_T // 8, 8), y_sorted, _T, 8)
    out = _comb_call(x2, y0, y1, w1, w2)
    return out.reshape(_B, _S, _D)


# speedup vs baseline: 2.3833x; 1.2218x over previous
import functools
import math

import jax
import jax.numpy as jnp
from jax.experimental import pallas as pl
from jax.experimental.pallas import tpu as pltpu
from jax.experimental.pallas import tpu_sc as plsc

_B, _S, _D = 1, 2048, 2048
_H, _KVH, _HD = 16, 4, 128
_E, _K, _F = 8, 2, 768
_EPS = 1e-6
_THETA = 10000.0
_T = _B * _S
_A = _T * _K
_BT = 256
_NBLK = _A // _BT + _E
_APAD = _NBLK * _BT
_BQ = 512
_BSA = 256
_SCALE = 1.0 / math.sqrt(_HD)
_F32 = jnp.float32
_BF16 = jnp.bfloat16
_BSQ = 512


def _rms(x, w):
    return x * jax.lax.rsqrt(jnp.mean(x * x, axis=-1, keepdims=True) + _EPS) * w


def _qkv_body(x_ref, ln1_ref, wq_ref, wk_ref, wv_ref, q_ref, k_ref, v_ref):
    x = x_ref[...]
    h = _rms(x, ln1_ref[...]).astype(_BF16)
    q_ref[...] = jnp.dot(h, wq_ref[...].astype(_BF16),
                         preferred_element_type=_F32)
    k_ref[...] = jnp.dot(h, wk_ref[...].astype(_BF16),
                         preferred_element_type=_F32)
    v_ref[...] = jnp.dot(h, wv_ref[...].astype(_BF16),
                         preferred_element_type=_F32)


def _qkv_call(x, ln1_w, wq, wk, wv):
    grid = (_S // _BSQ,)
    return pl.pallas_call(
        _qkv_body,
        grid=grid,
        in_specs=[
            pl.BlockSpec((_BSQ, _D), lambda i: (i, 0)),
            pl.BlockSpec((1, _D), lambda i: (0, 0)),
            pl.BlockSpec((_D, _H * _HD), lambda i: (0, 0)),
            pl.BlockSpec((_D, _KVH * _HD), lambda i: (0, 0)),
            pl.BlockSpec((_D, _KVH * _HD), lambda i: (0, 0)),
        ],
        out_specs=[
            pl.BlockSpec((_BSQ, _H * _HD), lambda i: (i, 0)),
            pl.BlockSpec((_BSQ, _KVH * _HD), lambda i: (i, 0)),
            pl.BlockSpec((_BSQ, _KVH * _HD), lambda i: (i, 0)),
        ],
        out_shape=[
            jax.ShapeDtypeStruct((_S, _H * _HD), _F32),
            jax.ShapeDtypeStruct((_S, _KVH * _HD), _F32),
            jax.ShapeDtypeStruct((_S, _KVH * _HD), _F32),
        ],
        compiler_params=pltpu.CompilerParams(
            vmem_limit_bytes=100 * 1024 * 1024),
    )(x, ln1_w.reshape(1, _D), wq, wk, wv)


def _prep_body(x_ref, nw_ref, o_ref):
    half = _HD // 2
    x = _rms(x_ref[...], nw_ref[...])
    pos = jax.lax.broadcasted_iota(jnp.int32, (_S, 1), 0).astype(_F32)
    inv = jnp.exp(jax.lax.broadcasted_iota(jnp.int32, (1, half), 1)
                  .astype(_F32) * (-math.log(_THETA) / half))
    f = pos * inv
    cos = jnp.cos(f)
    sin = jnp.sin(f)
    x1 = x[:, :half]
    x2 = x[:, half:]
    o_ref[0] = jnp.concatenate([x1 * cos - x2 * sin, x2 * cos + x1 * sin],
                               axis=-1).astype(_BF16)


def _prep_call(x2d, nw, nh):
    return pl.pallas_call(
        _prep_body,
        grid=(nh,),
        in_specs=[
            pl.BlockSpec((_S, _HD), lambda h: (0, h)),
            pl.BlockSpec((1, _HD), lambda h: (0, 0)),
        ],
        out_specs=pl.BlockSpec((1, _S, _HD), lambda h: (h, 0, 0)),
        out_shape=jax.ShapeDtypeStruct((nh, _S, _HD), _BF16),
        compiler_params=pltpu.CompilerParams(
            vmem_limit_bytes=100 * 1024 * 1024),
    )(x2d, nw.reshape(1, _HD))


def _attn_body(q_ref, k_ref, v_ref, o_ref):
    qb = pl.program_id(1)
    q = q_ref[0]

    def step(j, carry, masked):
        m, l, acc = carry
        k = k_ref[0, pl.ds(j * _BQ, _BQ), :]
        s = jax.lax.dot_general(q, k, (((1,), (1,)), ((), ())),
                                preferred_element_type=_F32) * _SCALE
        if masked:
            row = jax.lax.broadcasted_iota(jnp.int32, (_BQ, _BQ), 0)
            col = jax.lax.broadcasted_iota(jnp.int32, (_BQ, _BQ), 1)
            s = jnp.where(row >= col, s, -1e30)
        mj = jnp.max(s, axis=-1, keepdims=True)
        mn = jnp.maximum(m, mj)
        p = jnp.exp(s - mn)
        c = jnp.exp(m - mn)
        v = v_ref[0, pl.ds(j * _BQ, _BQ), :]
        acc = acc * c + jnp.dot(p.astype(_BF16), v,
                                preferred_element_type=_F32)
        l = l * c + jnp.sum(p, axis=-1, keepdims=True)
        return mn, l, acc

    init = (jnp.full((_BQ, 1), -1e30, _F32),
            jnp.zeros((_BQ, 1), _F32),
            jnp.zeros((_BQ, _HD), _F32))
    m, l, acc = jax.lax.fori_loop(
        0, qb, lambda j, cr: step(j, cr, False), init)
    m, l, acc = step(qb, (m, l, acc), True)
    o_ref[0] = (acc / l).astype(_BF16)


def _attn_call(q, k, v):
    rep = _H // _KVH
    grid = (_H, _S // _BQ)
    return pl.pallas_call(
        _attn_body,
        grid=grid,
        in_specs=[
            pl.BlockSpec((1, _BQ, _HD), lambda h, qb: (h, qb, 0)),
            pl.BlockSpec((1, _S, _HD), lambda h, qb: (h // rep, 0, 0)),
            pl.BlockSpec((1, _S, _HD), lambda h, qb: (h // rep, 0, 0)),
        ],
        out_specs=pl.BlockSpec((1, _BQ, _HD), lambda h, qb: (h, qb, 0)),
        out_shape=jax.ShapeDtypeStruct((_H, _S, _HD), _BF16),
        compiler_params=pltpu.CompilerParams(
            vmem_limit_bytes=100 * 1024 * 1024),
    )(q, k, v)


def _proj_body(o_ref, wo_ref, res_ref, out_ref):
    out_ref[...] = res_ref[...] + jnp.dot(o_ref[...],
                                          wo_ref[...].astype(_BF16),
                                          preferred_element_type=_F32)


def _proj_call(o, wo, res):
    grid = (_S // _BSQ,)
    return pl.pallas_call(
        _proj_body,
        grid=grid,
        in_specs=[
            pl.BlockSpec((_BSQ, _H * _HD), lambda i: (i, 0)),
            pl.BlockSpec((_H * _HD, _D), lambda i: (0, 0)),
            pl.BlockSpec((_BSQ, _D), lambda i: (i, 0)),
        ],
        out_specs=pl.BlockSpec((_BSQ, _D), lambda i: (i, 0)),
        out_shape=jax.ShapeDtypeStruct((_T, _D), _F32),
        compiler_params=pltpu.CompilerParams(
            vmem_limit_bytes=100 * 1024 * 1024),
    )(o, wo, res)


def _router_body(x_ref, ln2_ref, rw_ref, h_ref, w1_ref, w2_ref, i1_ref,
                 i2_ref):
    x = x_ref[...]
    h = _rms(x, ln2_ref[...])
    h_ref[...] = h
    logits = jnp.dot(h, rw_ref[...], preferred_element_type=_F32)
    m = jnp.max(logits, axis=-1, keepdims=True)
    p = jnp.exp(logits - m)
    p = p / jnp.sum(p, axis=-1, keepdims=True)
    ids = jax.lax.broadcasted_iota(jnp.int32, (_BSA, _E), 1)
    m1 = jnp.max(p, axis=-1, keepdims=True)
    i1 = jnp.min(jnp.where(p == m1, ids, _E), axis=-1, keepdims=True)
    pm = jnp.where(ids == i1, -1.0, p)
    m2 = jnp.max(pm, axis=-1, keepdims=True)
    i2 = jnp.min(jnp.where(pm == m2, ids, _E), axis=-1, keepdims=True)
    sw = m1 + m2
    w1_ref[...] = m1 / sw
    w2_ref[...] = m2 / sw
    i1_ref[...] = i1
    i2_ref[...] = i2


def _router_call(x, ln2_w, rw):
    grid = (_S // _BSA,)
    return pl.pallas_call(
        _router_body,
        grid=grid,
        in_specs=[
            pl.BlockSpec((_BSA, _D), lambda i: (i, 0)),
            pl.BlockSpec((1, _D), lambda i: (0, 0)),
            pl.BlockSpec((_D, _E), lambda i: (0, 0)),
        ],
        out_specs=[
            pl.BlockSpec((_BSA, _D), lambda i: (i, 0)),
            pl.BlockSpec((_BSA, 1), lambda i: (i, 0)),
            pl.BlockSpec((_BSA, 1), lambda i: (i, 0)),
            pl.BlockSpec((_BSA, 1), lambda i: (i, 0)),
            pl.BlockSpec((_BSA, 1), lambda i: (i, 0)),
        ],
        out_shape=[
            jax.ShapeDtypeStruct((_T, _D), _F32),
            jax.ShapeDtypeStruct((_T, 1), _F32),
            jax.ShapeDtypeStruct((_T, 1), _F32),
            jax.ShapeDtypeStruct((_T, 1), jnp.int32),
            jax.ShapeDtypeStruct((_T, 1), jnp.int32),
        ],
        compiler_params=pltpu.CompilerParams(
            vmem_limit_bytes=100 * 1024 * 1024),
    )(x, ln2_w.reshape(1, _D), rw)


def _moe_body(meta_ref, x_ref, wg_ref, wu_ref, wd_ref, y_ref,
              wg16_ref, wu16_ref, wd16_ref):
    b = pl.program_id(0)
    live = meta_ref[1, b] > 0
    changed = jnp.logical_or(
        b == 0, meta_ref[0, b] != meta_ref[0, jnp.maximum(b - 1, 0)])

    @pl.when(jnp.logical_and(live, changed))
    def _():
        wg16_ref[...] = wg_ref[0].astype(_BF16)
        wu16_ref[...] = wu_ref[0].astype(_BF16)
        wd16_ref[...] = wd_ref[0].astype(_BF16)

    @pl.when(live)
    def _():
        x = x_ref[...].astype(_BF16)
        g = jnp.dot(x, wg16_ref[...], preferred_element_type=_F32)
        u = jnp.dot(x, wu16_ref[...], preferred_element_type=_F32)
        a = (g * jax.lax.logistic(g) * u).astype(_BF16)
        y_ref[...] = jnp.dot(a, wd16_ref[...], preferred_element_type=_F32)

    @pl.when(jnp.logical_not(live))
    def _():
        y_ref[...] = jnp.zeros_like(y_ref)


def _moe_call(meta, x_sorted, wg, wu, wd):
    grid_spec = pltpu.PrefetchScalarGridSpec(
        num_scalar_prefetch=1,
        grid=(_NBLK,),
        in_specs=[
            pl.BlockSpec((_BT, _D), lambda b, m: (b, 0)),
            pl.BlockSpec((1, _D, _F), lambda b, m: (m[0, b], 0, 0)),
            pl.BlockSpec((1, _D, _F), lambda b, m: (m[0, b], 0, 0)),
            pl.BlockSpec((1, _F, _D), lambda b, m: (m[0, b], 0, 0)),
        ],
        out_specs=pl.BlockSpec((_BT, _D), lambda b, m: (b, 0)),
        scratch_shapes=[
            pltpu.VMEM((_D, _F), _BF16),
            pltpu.VMEM((_D, _F), _BF16),
            pltpu.VMEM((_F, _D), _BF16),
        ],
    )
    return pl.pallas_call(
        _moe_body,
        grid_spec=grid_spec,
        out_shape=jax.ShapeDtypeStruct((_APAD, _D), _F32),
        compiler_params=pltpu.CompilerParams(
            dimension_semantics=("arbitrary",),
            vmem_limit_bytes=110 * 1024 * 1024),
    )(meta, x_sorted, wg, wu, wd)


_NC, _NS = 2, 16
_NW = _NC * _NS
_DW = _D


def _sc_gather(idx2d, table, n_rows, chunk):
    per_w = n_rows // _NW
    n_ch = per_w // chunk
    mesh = plsc.VectorSubcoreMesh(core_axis_name="c", subcore_axis_name="s")

    @functools.partial(
        pl.kernel, mesh=mesh,
        out_type=jax.ShapeDtypeStruct((n_rows, _DW), _F32),
        scratch_types=[
            pltpu.VMEM((n_ch, chunk), jnp.int32),
            pltpu.VMEM((2, chunk, _DW), _F32),
            pltpu.SemaphoreType.DMA((2,)),
            pltpu.SemaphoreType.DMA((2,)),
        ],
    )
    def gath(idx_hbm, tab_hbm, out_hbm, idx_v, buf_v, gsem, wsem):
        wid = jax.lax.axis_index("s") * _NC + jax.lax.axis_index("c")
        base = wid * per_w
        pltpu.sync_copy(idx_hbm.at[pl.ds(wid * n_ch, n_ch)], idx_v)
        gh = {}
        wh = {}

        def fire(c):
            gh[c] = pltpu.async_copy(tab_hbm.at[idx_v.at[c]],
                                     buf_v.at[c % 2], gsem.at[c % 2])

        fire(0)
        for c in range(n_ch):
            if c + 1 < n_ch:
                if c - 1 >= 0:
                    wh[c - 1].wait()
                fire(c + 1)
            gh[c].wait()
            wh[c] = pltpu.async_copy(
                buf_v.at[c % 2],
                out_hbm.at[pl.ds(base + c * chunk, chunk)],
                wsem.at[c % 2])
        for c in range(max(0, n_ch - 2), n_ch):
            wh[c].wait()

    return gath(idx2d, table)


def _comb_body(x2_ref, y0_ref, y1_ref, w1_ref, w2_ref, out_ref):
    out_ref[...] = (x2_ref[...] + w1_ref[...] * y0_ref[...]
                    + w2_ref[...] * y1_ref[...])


def _comb_call(x2, y0, y1, w1, w2):
    grid = (_S // _BSQ,)
    return pl.pallas_call(
        _comb_body,
        grid=grid,
        in_specs=[
            pl.BlockSpec((_BSQ, _D), lambda i: (i, 0)),
            pl.BlockSpec((_BSQ, _D), lambda i: (i, 0)),
            pl.BlockSpec((_BSQ, _D), lambda i: (i, 0)),
            pl.BlockSpec((_BSQ, 1), lambda i: (i, 0)),
            pl.BlockSpec((_BSQ, 1), lambda i: (i, 0)),
        ],
        out_specs=pl.BlockSpec((_BSQ, _D), lambda i: (i, 0)),
        out_shape=jax.ShapeDtypeStruct((_T, _D), _F32),
        compiler_params=pltpu.CompilerParams(
            vmem_limit_bytes=100 * 1024 * 1024),
    )(x2, y0, y1, w1, w2)


def _dispatch(i1, i2):
    topi = jnp.concatenate([i1, i2], axis=1)
    flat_e = topi.reshape(_A)
    onehot = (flat_e[:, None] == jnp.arange(_E)[None, :]).astype(jnp.int32)
    rank = jnp.take_along_axis(jnp.cumsum(onehot, axis=0) - onehot,
                               flat_e[:, None], axis=1)[:, 0]
    counts = jnp.sum(onehot, axis=0)
    nblk_e = (counts + _BT - 1) // _BT
    end_blk = jnp.cumsum(nblk_e)
    used = end_blk[-1]
    start_row = jnp.concatenate(
        [jnp.zeros((1,), jnp.int32), jnp.cumsum(nblk_e * _BT)[:-1]])
    pos = start_row[flat_e] + rank
    tok = jnp.arange(_A, dtype=jnp.int32) // _K
    tok_sorted = (jnp.arange(_APAD, dtype=jnp.int32) % _T).at[pos].set(tok)
    bidx = jnp.arange(_NBLK, dtype=jnp.int32)
    eob = jnp.sum((bidx[:, None] >= end_blk[None, :]).astype(jnp.int32),
                  axis=1)
    is_real = (bidx < used).astype(jnp.int32)
    eob_last = jnp.sum((end_blk <= used - 1).astype(jnp.int32))
    eob = jnp.where(is_real > 0, eob, eob_last)
    meta = jnp.stack([eob, is_real]).astype(jnp.int32)
    return tok_sorted, meta, pos.reshape(_T, _K).astype(jnp.int32)


def kernel(hidden_states, ln1_w, Wq, Wk, Wv, q_norm_w, k_norm_w, Wo, ln2_w,
           router_W, W_gate, W_up, W_down):
    x = hidden_states.reshape(_T, _D)
    q2, k2, v2 = _qkv_call(x, ln1_w, Wq, Wk, Wv)
    q = _prep_call(q2, q_norm_w, _H)
    k = _prep_call(k2, k_norm_w, _KVH)
    v = v2.reshape(_S, _KVH, _HD).transpose(1, 0, 2).astype(_BF16)
    o = _attn_call(q, k, v)
    x2 = _proj_call(o.transpose(1, 0, 2).reshape(_S, _H * _HD), Wo, x)
    h2, w1, w2, i1, i2 = _router_call(x2, ln2_w, router_W)
    tok_sorted, meta, posr = _dispatch(i1, i2)
    x_sorted = _sc_gather(tok_sorted.reshape(_APAD // 8, 8), h2, _APAD, 8)
    y_sorted = _moe_call(meta, x_sorted, W_gate, W_up, W_down)
    y0 = _sc_gather(posr[:, 0].reshape(_T // 8, 8), y_sorted, _T, 8)
    y1 = _sc_gather(posr[:, 1].reshape(_T // 8, 8), y_sorted, _T, 8)
    out = _comb_call(x2, y0, y1, w1, w2)
    return out.reshape(_B, _S, _D)


# speedup vs baseline: 2.4216x; 1.0161x over previous
import functools
import math

import jax
import jax.numpy as jnp
from jax.experimental import pallas as pl
from jax.experimental.pallas import tpu as pltpu
from jax.experimental.pallas import tpu_sc as plsc

_B, _S, _D = 1, 2048, 2048
_H, _KVH, _HD = 16, 4, 128
_E, _K, _F = 8, 2, 768
_EPS = 1e-6
_THETA = 10000.0
_T = _B * _S
_A = _T * _K
_BT = 256
_NBLK = _A // _BT + _E
_APAD = _NBLK * _BT
_BQ = 512
_BSA = 256
_SCALE = 1.0 / math.sqrt(_HD)
_F32 = jnp.float32
_BF16 = jnp.bfloat16
_BSQ = 512


def _rms(x, w):
    return x * jax.lax.rsqrt(jnp.mean(x * x, axis=-1, keepdims=True) + _EPS) * w


def _qkv_body(x_ref, ln1_ref, wq_ref, wk_ref, wv_ref, q_ref, k_ref, v_ref):
    x = x_ref[...]
    h = _rms(x, ln1_ref[...]).astype(_BF16)
    q_ref[...] = jnp.dot(h, wq_ref[...].astype(_BF16),
                         preferred_element_type=_F32)
    k_ref[...] = jnp.dot(h, wk_ref[...].astype(_BF16),
                         preferred_element_type=_F32)
    v_ref[...] = jnp.dot(h, wv_ref[...].astype(_BF16),
                         preferred_element_type=_F32)


def _qkv_call(x, ln1_w, wq, wk, wv):
    grid = (_S // _BSQ,)
    return pl.pallas_call(
        _qkv_body,
        grid=grid,
        in_specs=[
            pl.BlockSpec((_BSQ, _D), lambda i: (i, 0)),
            pl.BlockSpec((1, _D), lambda i: (0, 0)),
            pl.BlockSpec((_D, _H * _HD), lambda i: (0, 0)),
            pl.BlockSpec((_D, _KVH * _HD), lambda i: (0, 0)),
            pl.BlockSpec((_D, _KVH * _HD), lambda i: (0, 0)),
        ],
        out_specs=[
            pl.BlockSpec((_BSQ, _H * _HD), lambda i: (i, 0)),
            pl.BlockSpec((_BSQ, _KVH * _HD), lambda i: (i, 0)),
            pl.BlockSpec((_BSQ, _KVH * _HD), lambda i: (i, 0)),
        ],
        out_shape=[
            jax.ShapeDtypeStruct((_S, _H * _HD), _F32),
            jax.ShapeDtypeStruct((_S, _KVH * _HD), _F32),
            jax.ShapeDtypeStruct((_S, _KVH * _HD), _F32),
        ],
        compiler_params=pltpu.CompilerParams(
            vmem_limit_bytes=100 * 1024 * 1024),
    )(x, ln1_w.reshape(1, _D), wq, wk, wv)


def _prep_body(x_ref, nw_ref, o_ref):
    half = _HD // 2
    x = _rms(x_ref[...], nw_ref[...])
    pos = jax.lax.broadcasted_iota(jnp.int32, (_S, 1), 0).astype(_F32)
    inv = jnp.exp(jax.lax.broadcasted_iota(jnp.int32, (1, half), 1)
                  .astype(_F32) * (-math.log(_THETA) / half))
    f = pos * inv
    cos = jnp.cos(f)
    sin = jnp.sin(f)
    x1 = x[:, :half]
    x2 = x[:, half:]
    o_ref[0] = jnp.concatenate([x1 * cos - x2 * sin, x2 * cos + x1 * sin],
                               axis=-1).astype(_BF16)


def _prep_call(x2d, nw, nh):
    return pl.pallas_call(
        _prep_body,
        grid=(nh,),
        in_specs=[
            pl.BlockSpec((_S, _HD), lambda h: (0, h)),
            pl.BlockSpec((1, _HD), lambda h: (0, 0)),
        ],
        out_specs=pl.BlockSpec((1, _S, _HD), lambda h: (h, 0, 0)),
        out_shape=jax.ShapeDtypeStruct((nh, _S, _HD), _BF16),
        compiler_params=pltpu.CompilerParams(
            vmem_limit_bytes=100 * 1024 * 1024),
    )(x2d, nw.reshape(1, _HD))


def _attn_body(q_ref, k_ref, v_ref, o_ref):
    qb = pl.program_id(1)
    q = q_ref[0]

    def step(j, carry, masked):
        m, l, acc = carry
        k = k_ref[0, pl.ds(j * _BQ, _BQ), :]
        s = jax.lax.dot_general(q, k, (((1,), (1,)), ((), ())),
                                preferred_element_type=_F32) * _SCALE
        if masked:
            row = jax.lax.broadcasted_iota(jnp.int32, (_BQ, _BQ), 0)
            col = jax.lax.broadcasted_iota(jnp.int32, (_BQ, _BQ), 1)
            s = jnp.where(row >= col, s, -1e30)
        mj = jnp.max(s, axis=-1, keepdims=True)
        mn = jnp.maximum(m, mj)
        p = jnp.exp(s - mn)
        c = jnp.exp(m - mn)
        v = v_ref[0, pl.ds(j * _BQ, _BQ), :]
        acc = acc * c + jnp.dot(p.astype(_BF16), v,
                                preferred_element_type=_F32)
        l = l * c + jnp.sum(p, axis=-1, keepdims=True)
        return mn, l, acc

    init = (jnp.full((_BQ, 1), -1e30, _F32),
            jnp.zeros((_BQ, 1), _F32),
            jnp.zeros((_BQ, _HD), _F32))
    m, l, acc = jax.lax.fori_loop(
        0, qb, lambda j, cr: step(j, cr, False), init)
    m, l, acc = step(qb, (m, l, acc), True)
    o_ref[0] = (acc / l).astype(_BF16)


def _attn_call(q, k, v):
    rep = _H // _KVH
    grid = (_H, _S // _BQ)
    return pl.pallas_call(
        _attn_body,
        grid=grid,
        in_specs=[
            pl.BlockSpec((1, _BQ, _HD), lambda h, qb: (h, qb, 0)),
            pl.BlockSpec((1, _S, _HD), lambda h, qb: (h // rep, 0, 0)),
            pl.BlockSpec((1, _S, _HD), lambda h, qb: (h // rep, 0, 0)),
        ],
        out_specs=pl.BlockSpec((1, _BQ, _HD), lambda h, qb: (h, qb, 0)),
        out_shape=jax.ShapeDtypeStruct((_H, _S, _HD), _BF16),
        compiler_params=pltpu.CompilerParams(
            vmem_limit_bytes=100 * 1024 * 1024),
    )(q, k, v)


def _proj_body(o_ref, wo_ref, res_ref, out_ref):
    out_ref[...] = res_ref[...] + jnp.dot(o_ref[...],
                                          wo_ref[...].astype(_BF16),
                                          preferred_element_type=_F32)


def _proj_call(o, wo, res):
    grid = (_S // _BSQ,)
    return pl.pallas_call(
        _proj_body,
        grid=grid,
        in_specs=[
            pl.BlockSpec((_BSQ, _H * _HD), lambda i: (i, 0)),
            pl.BlockSpec((_H * _HD, _D), lambda i: (0, 0)),
            pl.BlockSpec((_BSQ, _D), lambda i: (i, 0)),
        ],
        out_specs=pl.BlockSpec((_BSQ, _D), lambda i: (i, 0)),
        out_shape=jax.ShapeDtypeStruct((_T, _D), _F32),
        compiler_params=pltpu.CompilerParams(
            vmem_limit_bytes=100 * 1024 * 1024),
    )(o, wo, res)


def _router_body(x_ref, ln2_ref, rw_ref, h_ref, w1_ref, w2_ref, i1_ref,
                 i2_ref):
    x = x_ref[...]
    h = _rms(x, ln2_ref[...])
    h_ref[...] = h
    logits = jnp.dot(h, rw_ref[...], preferred_element_type=_F32)
    m = jnp.max(logits, axis=-1, keepdims=True)
    p = jnp.exp(logits - m)
    p = p / jnp.sum(p, axis=-1, keepdims=True)
    ids = jax.lax.broadcasted_iota(jnp.int32, (_BSA, _E), 1)
    m1 = jnp.max(p, axis=-1, keepdims=True)
    i1 = jnp.min(jnp.where(p == m1, ids, _E), axis=-1, keepdims=True)
    pm = jnp.where(ids == i1, -1.0, p)
    m2 = jnp.max(pm, axis=-1, keepdims=True)
    i2 = jnp.min(jnp.where(pm == m2, ids, _E), axis=-1, keepdims=True)
    sw = m1 + m2
    w1_ref[...] = m1 / sw
    w2_ref[...] = m2 / sw
    i1_ref[...] = i1
    i2_ref[...] = i2


def _router_call(x, ln2_w, rw):
    grid = (_S // _BSA,)
    return pl.pallas_call(
        _router_body,
        grid=grid,
        in_specs=[
            pl.BlockSpec((_BSA, _D), lambda i: (i, 0)),
            pl.BlockSpec((1, _D), lambda i: (0, 0)),
            pl.BlockSpec((_D, _E), lambda i: (0, 0)),
        ],
        out_specs=[
            pl.BlockSpec((_BSA, _D), lambda i: (i, 0)),
            pl.BlockSpec((_BSA, 1), lambda i: (i, 0)),
            pl.BlockSpec((_BSA, 1), lambda i: (i, 0)),
            pl.BlockSpec((_BSA, 1), lambda i: (i, 0)),
            pl.BlockSpec((_BSA, 1), lambda i: (i, 0)),
        ],
        out_shape=[
            jax.ShapeDtypeStruct((_T, _D), _F32),
            jax.ShapeDtypeStruct((_T, 1), _F32),
            jax.ShapeDtypeStruct((_T, 1), _F32),
            jax.ShapeDtypeStruct((_T, 1), jnp.int32),
            jax.ShapeDtypeStruct((_T, 1), jnp.int32),
        ],
        compiler_params=pltpu.CompilerParams(
            vmem_limit_bytes=100 * 1024 * 1024),
    )(x, ln2_w.reshape(1, _D), rw)


def _moe_body(meta_ref, x_ref, wg_ref, wu_ref, wd_ref, y_ref,
              wg16_ref, wu16_ref, wd16_ref):
    b = pl.program_id(0)
    live = meta_ref[1, b] > 0
    changed = jnp.logical_or(
        b == 0, meta_ref[0, b] != meta_ref[0, jnp.maximum(b - 1, 0)])

    @pl.when(jnp.logical_and(live, changed))
    def _():
        wg16_ref[...] = wg_ref[0].astype(_BF16)
        wu16_ref[...] = wu_ref[0].astype(_BF16)
        wd16_ref[...] = wd_ref[0].astype(_BF16)

    @pl.when(live)
    def _():
        x = x_ref[...].astype(_BF16)
        g = jnp.dot(x, wg16_ref[...], preferred_element_type=_F32)
        u = jnp.dot(x, wu16_ref[...], preferred_element_type=_F32)
        a = (g * jax.lax.logistic(g) * u).astype(_BF16)
        y_ref[...] = jnp.dot(a, wd16_ref[...], preferred_element_type=_F32)

    @pl.when(jnp.logical_not(live))
    def _():
        y_ref[...] = jnp.zeros_like(y_ref)


def _moe_call(meta, x_sorted, wg, wu, wd):
    grid_spec = pltpu.PrefetchScalarGridSpec(
        num_scalar_prefetch=1,
        grid=(_NBLK,),
        in_specs=[
            pl.BlockSpec((_BT, _D), lambda b, m: (b, 0)),
            pl.BlockSpec((1, _D, _F), lambda b, m: (m[0, b], 0, 0)),
            pl.BlockSpec((1, _D, _F), lambda b, m: (m[0, b], 0, 0)),
            pl.BlockSpec((1, _F, _D), lambda b, m: (m[0, b], 0, 0)),
        ],
        out_specs=pl.BlockSpec((_BT, _D), lambda b, m: (b, 0)),
        scratch_shapes=[
            pltpu.VMEM((_D, _F), _BF16),
            pltpu.VMEM((_D, _F), _BF16),
            pltpu.VMEM((_F, _D), _BF16),
        ],
    )
    return pl.pallas_call(
        _moe_body,
        grid_spec=grid_spec,
        out_shape=jax.ShapeDtypeStruct((_APAD, _D), _F32),
        compiler_params=pltpu.CompilerParams(
            dimension_semantics=("arbitrary",),
            vmem_limit_bytes=110 * 1024 * 1024),
    )(meta, x_sorted, wg, wu, wd)


_NC, _NS = 2, 16
_NW = _NC * _NS
_DW = _D


def _sc_gather(idx2d, table, n_rows, chunk):
    per_w = n_rows // _NW
    n_ch = per_w // chunk
    mesh = plsc.VectorSubcoreMesh(core_axis_name="c", subcore_axis_name="s")

    @functools.partial(
        pl.kernel, mesh=mesh,
        out_type=jax.ShapeDtypeStruct((n_rows, _DW), _F32),
        scratch_types=[
            pltpu.VMEM((n_ch, chunk), jnp.int32),
            pltpu.VMEM((2, chunk, _DW), _F32),
            pltpu.SemaphoreType.DMA((2,)),
            pltpu.SemaphoreType.DMA((2,)),
        ],
    )
    def gath(idx_hbm, tab_hbm, out_hbm, idx_v, buf_v, gsem, wsem):
        wid = jax.lax.axis_index("s") * _NC + jax.lax.axis_index("c")
        base = wid * per_w
        pltpu.sync_copy(idx_hbm.at[pl.ds(wid * n_ch, n_ch)], idx_v)
        gh = {}
        wh = {}

        def fire(c):
            gh[c] = pltpu.async_copy(tab_hbm.at[idx_v.at[c]],
                                     buf_v.at[c % 2], gsem.at[c % 2])

        fire(0)
        for c in range(n_ch):
            if c + 1 < n_ch:
                if c - 1 >= 0:
                    wh[c - 1].wait()
                fire(c + 1)
            gh[c].wait()
            wh[c] = pltpu.async_copy(
                buf_v.at[c % 2],
                out_hbm.at[pl.ds(base + c * chunk, chunk)],
                wsem.at[c % 2])
        for c in range(max(0, n_ch - 2), n_ch):
            wh[c].wait()

    return gath(idx2d, table)


def _comb_body(x2_ref, y0_ref, y1_ref, w1_ref, w2_ref, out_ref):
    out_ref[...] = (x2_ref[...] + w1_ref[...] * y0_ref[...]
                    + w2_ref[...] * y1_ref[...])


def _comb_call(x2, y01, w1, w2):
    grid = (_S // _BSQ,)
    nqb = _S // _BSQ
    return pl.pallas_call(
        _comb_body,
        grid=grid,
        in_specs=[
            pl.BlockSpec((_BSQ, _D), lambda i: (i, 0)),
            pl.BlockSpec((_BSQ, _D), lambda i: (i, 0)),
            pl.BlockSpec((_BSQ, _D), lambda i: (i + nqb, 0)),
            pl.BlockSpec((_BSQ, 1), lambda i: (i, 0)),
            pl.BlockSpec((_BSQ, 1), lambda i: (i, 0)),
        ],
        out_specs=pl.BlockSpec((_BSQ, _D), lambda i: (i, 0)),
        out_shape=jax.ShapeDtypeStruct((_T, _D), _F32),
        compiler_params=pltpu.CompilerParams(
            vmem_limit_bytes=100 * 1024 * 1024),
    )(x2, y01, y01, w1, w2)


def _dispatch(i1, i2):
    topi = jnp.concatenate([i1, i2], axis=1)
    flat_e = topi.reshape(_A)
    onehot = (flat_e[:, None] == jnp.arange(_E)[None, :]).astype(jnp.int32)
    rank = jnp.take_along_axis(jnp.cumsum(onehot, axis=0) - onehot,
                               flat_e[:, None], axis=1)[:, 0]
    counts = jnp.sum(onehot, axis=0)
    nblk_e = (counts + _BT - 1) // _BT
    end_blk = jnp.cumsum(nblk_e)
    used = end_blk[-1]
    start_row = jnp.concatenate(
        [jnp.zeros((1,), jnp.int32), jnp.cumsum(nblk_e * _BT)[:-1]])
    pos = start_row[flat_e] + rank
    tok = jnp.arange(_A, dtype=jnp.int32) // _K
    tok_sorted = (jnp.arange(_APAD, dtype=jnp.int32) % _T).at[pos].set(tok)
    bidx = jnp.arange(_NBLK, dtype=jnp.int32)
    eob = jnp.sum((bidx[:, None] >= end_blk[None, :]).astype(jnp.int32),
                  axis=1)
    is_real = (bidx < used).astype(jnp.int32)
    eob_last = jnp.sum((end_blk <= used - 1).astype(jnp.int32))
    eob = jnp.where(is_real > 0, eob, eob_last)
    meta = jnp.stack([eob, is_real]).astype(jnp.int32)
    return tok_sorted, meta, pos.reshape(_T, _K).astype(jnp.int32)


def kernel(hidden_states, ln1_w, Wq, Wk, Wv, q_norm_w, k_norm_w, Wo, ln2_w,
           router_W, W_gate, W_up, W_down):
    x = hidden_states.reshape(_T, _D)
    q2, k2, v2 = _qkv_call(x, ln1_w, Wq, Wk, Wv)
    q = _prep_call(q2, q_norm_w, _H)
    k = _prep_call(k2, k_norm_w, _KVH)
    v = v2.reshape(_S, _KVH, _HD).transpose(1, 0, 2).astype(_BF16)
    o = _attn_call(q, k, v)
    x2 = _proj_call(o.transpose(1, 0, 2).reshape(_S, _H * _HD), Wo, x)
    h2, w1, w2, i1, i2 = _router_call(x2, ln2_w, router_W)
    tok_sorted, meta, posr = _dispatch(i1, i2)
    x_sorted = _sc_gather(tok_sorted.reshape(_APAD // 24, 24), h2, _APAD, 24)
    y_sorted = _moe_call(meta, x_sorted, W_gate, W_up, W_down)
    pos01 = posr.transpose(1, 0).reshape(2 * _T // 16, 16)
    y01 = _sc_gather(pos01, y_sorted, 2 * _T, 16)
    out = _comb_call(x2, y01, w1, w2)
    return out.reshape(_B, _S, _D)


# speedup vs baseline: 2.4680x; 1.0192x over previous
import functools
import math

import jax
import jax.numpy as jnp
from jax.experimental import pallas as pl
from jax.experimental.pallas import tpu as pltpu
from jax.experimental.pallas import tpu_sc as plsc

_B, _S, _D = 1, 2048, 2048
_H, _KVH, _HD = 16, 4, 128
_E, _K, _F = 8, 2, 768
_EPS = 1e-6
_THETA = 10000.0
_T = _B * _S
_A = _T * _K
_BT = 256
_NBLK = _A // _BT + _E
_APAD = _NBLK * _BT
_BQ = 512
_BSA = 256
_SCALE = 1.0 / math.sqrt(_HD)
_F32 = jnp.float32
_BF16 = jnp.bfloat16
_BSQ = 512


def _rms(x, w):
    return x * jax.lax.rsqrt(jnp.mean(x * x, axis=-1, keepdims=True) + _EPS) * w


def _qkv_body(x_ref, ln1_ref, wq_ref, wk_ref, wv_ref, q_ref, k_ref, v_ref):
    x = x_ref[...]
    h = _rms(x, ln1_ref[...]).astype(_BF16)
    q_ref[...] = jnp.dot(h, wq_ref[...].astype(_BF16),
                         preferred_element_type=_F32)
    k_ref[...] = jnp.dot(h, wk_ref[...].astype(_BF16),
                         preferred_element_type=_F32)
    v_ref[...] = jnp.dot(h, wv_ref[...].astype(_BF16),
                         preferred_element_type=_F32)


def _qkv_call(x, ln1_w, wq, wk, wv):
    grid = (_S // _BSQ,)
    return pl.pallas_call(
        _qkv_body,
        grid=grid,
        in_specs=[
            pl.BlockSpec((_BSQ, _D), lambda i: (i, 0)),
            pl.BlockSpec((1, _D), lambda i: (0, 0)),
            pl.BlockSpec((_D, _H * _HD), lambda i: (0, 0)),
            pl.BlockSpec((_D, _KVH * _HD), lambda i: (0, 0)),
            pl.BlockSpec((_D, _KVH * _HD), lambda i: (0, 0)),
        ],
        out_specs=[
            pl.BlockSpec((_BSQ, _H * _HD), lambda i: (i, 0)),
            pl.BlockSpec((_BSQ, _KVH * _HD), lambda i: (i, 0)),
            pl.BlockSpec((_BSQ, _KVH * _HD), lambda i: (i, 0)),
        ],
        out_shape=[
            jax.ShapeDtypeStruct((_S, _H * _HD), _F32),
            jax.ShapeDtypeStruct((_S, _KVH * _HD), _F32),
            jax.ShapeDtypeStruct((_S, _KVH * _HD), _F32),
        ],
        compiler_params=pltpu.CompilerParams(
            vmem_limit_bytes=100 * 1024 * 1024),
    )(x, ln1_w.reshape(1, _D), wq, wk, wv)


def _prep_body(x_ref, nw_ref, o_ref):
    half = _HD // 2
    x = _rms(x_ref[...], nw_ref[...])
    pos = jax.lax.broadcasted_iota(jnp.int32, (_S, 1), 0).astype(_F32)
    inv = jnp.exp(jax.lax.broadcasted_iota(jnp.int32, (1, half), 1)
                  .astype(_F32) * (-math.log(_THETA) / half))
    f = pos * inv
    cos = jnp.cos(f)
    sin = jnp.sin(f)
    x1 = x[:, :half]
    x2 = x[:, half:]
    o_ref[0] = jnp.concatenate([x1 * cos - x2 * sin, x2 * cos + x1 * sin],
                               axis=-1).astype(_BF16)


def _prep_call(x2d, nw, nh):
    return pl.pallas_call(
        _prep_body,
        grid=(nh,),
        in_specs=[
            pl.BlockSpec((_S, _HD), lambda h: (0, h)),
            pl.BlockSpec((1, _HD), lambda h: (0, 0)),
        ],
        out_specs=pl.BlockSpec((1, _S, _HD), lambda h: (h, 0, 0)),
        out_shape=jax.ShapeDtypeStruct((nh, _S, _HD), _BF16),
        compiler_params=pltpu.CompilerParams(
            vmem_limit_bytes=100 * 1024 * 1024),
    )(x2d, nw.reshape(1, _HD))


def _attn_body(q_ref, k_ref, v_ref, o_ref):
    qb = pl.program_id(1)
    q = q_ref[0]

    def step(j, carry, masked):
        m, l, acc = carry
        k = k_ref[0, pl.ds(j * _BQ, _BQ), :]
        s = jax.lax.dot_general(q, k, (((1,), (1,)), ((), ())),
                                preferred_element_type=_F32) * _SCALE
        if masked:
            row = jax.lax.broadcasted_iota(jnp.int32, (_BQ, _BQ), 0)
            col = jax.lax.broadcasted_iota(jnp.int32, (_BQ, _BQ), 1)
            s = jnp.where(row >= col, s, -1e30)
        mj = jnp.max(s, axis=-1, keepdims=True)
        mn = jnp.maximum(m, mj)
        p = jnp.exp(s - mn)
        c = jnp.exp(m - mn)
        v = v_ref[0, pl.ds(j * _BQ, _BQ), :]
        acc = acc * c + jnp.dot(p.astype(_BF16), v,
                                preferred_element_type=_F32)
        l = l * c + jnp.sum(p, axis=-1, keepdims=True)
        return mn, l, acc

    init = (jnp.full((_BQ, 1), -1e30, _F32),
            jnp.zeros((_BQ, 1), _F32),
            jnp.zeros((_BQ, _HD), _F32))
    m, l, acc = jax.lax.fori_loop(
        0, qb, lambda j, cr: step(j, cr, False), init)
    m, l, acc = step(qb, (m, l, acc), True)
    o_ref[0] = (acc / l).astype(_BF16)


def _attn_call(q, k, v):
    rep = _H // _KVH
    grid = (_H, _S // _BQ)
    return pl.pallas_call(
        _attn_body,
        grid=grid,
        in_specs=[
            pl.BlockSpec((1, _BQ, _HD), lambda h, qb: (h, qb, 0)),
            pl.BlockSpec((1, _S, _HD), lambda h, qb: (h // rep, 0, 0)),
            pl.BlockSpec((1, _S, _HD), lambda h, qb: (h // rep, 0, 0)),
        ],
        out_specs=pl.BlockSpec((1, _BQ, _HD), lambda h, qb: (h, qb, 0)),
        out_shape=jax.ShapeDtypeStruct((_H, _S, _HD), _BF16),
        compiler_params=pltpu.CompilerParams(
            vmem_limit_bytes=100 * 1024 * 1024),
    )(q, k, v)


def _proj_body(o_ref, wo_ref, res_ref, out_ref):
    out_ref[...] = res_ref[...] + jnp.dot(o_ref[...],
                                          wo_ref[...].astype(_BF16),
                                          preferred_element_type=_F32)


def _proj_call(o, wo, res):
    grid = (_S // _BSQ,)
    return pl.pallas_call(
        _proj_body,
        grid=grid,
        in_specs=[
            pl.BlockSpec((_BSQ, _H * _HD), lambda i: (i, 0)),
            pl.BlockSpec((_H * _HD, _D), lambda i: (0, 0)),
            pl.BlockSpec((_BSQ, _D), lambda i: (i, 0)),
        ],
        out_specs=pl.BlockSpec((_BSQ, _D), lambda i: (i, 0)),
        out_shape=jax.ShapeDtypeStruct((_T, _D), _F32),
        compiler_params=pltpu.CompilerParams(
            vmem_limit_bytes=100 * 1024 * 1024),
    )(o, wo, res)


def _router_body(x_ref, ln2_ref, rw_ref, h_ref, w1_ref, w2_ref, i1_ref,
                 i2_ref):
    x = x_ref[...]
    h = _rms(x, ln2_ref[...])
    h_ref[...] = h
    logits = jnp.dot(h, rw_ref[...], preferred_element_type=_F32)
    m = jnp.max(logits, axis=-1, keepdims=True)
    p = jnp.exp(logits - m)
    p = p / jnp.sum(p, axis=-1, keepdims=True)
    ids = jax.lax.broadcasted_iota(jnp.int32, (_BSA, _E), 1)
    m1 = jnp.max(p, axis=-1, keepdims=True)
    i1 = jnp.min(jnp.where(p == m1, ids, _E), axis=-1, keepdims=True)
    pm = jnp.where(ids == i1, -1.0, p)
    m2 = jnp.max(pm, axis=-1, keepdims=True)
    i2 = jnp.min(jnp.where(pm == m2, ids, _E), axis=-1, keepdims=True)
    sw = m1 + m2
    w1_ref[...] = m1 / sw
    w2_ref[...] = m2 / sw
    i1_ref[...] = i1
    i2_ref[...] = i2


def _router_call(x, ln2_w, rw):
    grid = (_S // _BSA,)
    return pl.pallas_call(
        _router_body,
        grid=grid,
        in_specs=[
            pl.BlockSpec((_BSA, _D), lambda i: (i, 0)),
            pl.BlockSpec((1, _D), lambda i: (0, 0)),
            pl.BlockSpec((_D, _E), lambda i: (0, 0)),
        ],
        out_specs=[
            pl.BlockSpec((_BSA, _D), lambda i: (i, 0)),
            pl.BlockSpec((_BSA, 1), lambda i: (i, 0)),
            pl.BlockSpec((_BSA, 1), lambda i: (i, 0)),
            pl.BlockSpec((_BSA, 1), lambda i: (i, 0)),
            pl.BlockSpec((_BSA, 1), lambda i: (i, 0)),
        ],
        out_shape=[
            jax.ShapeDtypeStruct((_T, _D), _F32),
            jax.ShapeDtypeStruct((_T, 1), _F32),
            jax.ShapeDtypeStruct((_T, 1), _F32),
            jax.ShapeDtypeStruct((_T, 1), jnp.int32),
            jax.ShapeDtypeStruct((_T, 1), jnp.int32),
        ],
        compiler_params=pltpu.CompilerParams(
            vmem_limit_bytes=100 * 1024 * 1024),
    )(x, ln2_w.reshape(1, _D), rw)


def _moe_body(meta_ref, x_ref, wg_ref, wu_ref, wd_ref, y_ref,
              wg16_ref, wu16_ref, wd16_ref):
    b = pl.program_id(0)
    live = meta_ref[1, b] > 0
    changed = jnp.logical_or(
        b == 0, meta_ref[0, b] != meta_ref[0, jnp.maximum(b - 1, 0)])

    @pl.when(jnp.logical_and(live, changed))
    def _():
        wg16_ref[...] = wg_ref[0].astype(_BF16)
        wu16_ref[...] = wu_ref[0].astype(_BF16)
        wd16_ref[...] = wd_ref[0].astype(_BF16)

    @pl.when(live)
    def _():
        x = x_ref[...].astype(_BF16)
        g = jnp.dot(x, wg16_ref[...], preferred_element_type=_F32)
        u = jnp.dot(x, wu16_ref[...], preferred_element_type=_F32)
        a = (g * jax.lax.logistic(g) * u).astype(_BF16)
        y_ref[...] = jnp.dot(a, wd16_ref[...], preferred_element_type=_F32)

    @pl.when(jnp.logical_not(live))
    def _():
        y_ref[...] = jnp.zeros_like(y_ref)


def _moe_call(meta, x_sorted, wg, wu, wd):
    grid_spec = pltpu.PrefetchScalarGridSpec(
        num_scalar_prefetch=1,
        grid=(_NBLK,),
        in_specs=[
            pl.BlockSpec((_BT, _D), lambda b, m: (b, 0)),
            pl.BlockSpec((1, _D, _F), lambda b, m: (m[0, b], 0, 0)),
            pl.BlockSpec((1, _D, _F), lambda b, m: (m[0, b], 0, 0)),
            pl.BlockSpec((1, _F, _D), lambda b, m: (m[0, b], 0, 0)),
        ],
        out_specs=pl.BlockSpec((_BT, _D), lambda b, m: (b, 0)),
        scratch_shapes=[
            pltpu.VMEM((_D, _F), _BF16),
            pltpu.VMEM((_D, _F), _BF16),
            pltpu.VMEM((_F, _D), _BF16),
        ],
    )
    return pl.pallas_call(
        _moe_body,
        grid_spec=grid_spec,
        out_shape=jax.ShapeDtypeStruct((_APAD, _D), _F32),
        compiler_params=pltpu.CompilerParams(
            dimension_semantics=("arbitrary",),
            vmem_limit_bytes=110 * 1024 * 1024),
    )(meta, x_sorted, wg, wu, wd)


_NC, _NS = 2, 16
_NW = _NC * _NS
_DW = _D


def _sc_gather(idx2d, table, n_rows, chunk):
    per_w = n_rows // _NW
    n_ch = per_w // chunk
    mesh = plsc.VectorSubcoreMesh(core_axis_name="c", subcore_axis_name="s")

    @functools.partial(
        pl.kernel, mesh=mesh,
        out_type=jax.ShapeDtypeStruct((n_rows, _DW), _F32),
        scratch_types=[
            pltpu.VMEM((n_ch, chunk), jnp.int32),
            pltpu.VMEM((2, chunk, _DW), _F32),
            pltpu.SemaphoreType.DMA((2,)),
            pltpu.SemaphoreType.DMA((2,)),
        ],
    )
    def gath(idx_hbm, tab_hbm, out_hbm, idx_v, buf_v, gsem, wsem):
        wid = jax.lax.axis_index("s") * _NC + jax.lax.axis_index("c")
        base = wid * per_w
        pltpu.sync_copy(idx_hbm.at[pl.ds(wid * n_ch, n_ch)], idx_v)
        gh = {}
        wh = {}

        def fire(c):
            gh[c] = pltpu.async_copy(tab_hbm.at[idx_v.at[c]],
                                     buf_v.at[c % 2], gsem.at[c % 2])

        fire(0)
        for c in range(n_ch):
            if c + 1 < n_ch:
                if c - 1 >= 0:
                    wh[c - 1].wait()
                fire(c + 1)
            gh[c].wait()
            wh[c] = pltpu.async_copy(
                buf_v.at[c % 2],
                out_hbm.at[pl.ds(base + c * chunk, chunk)],
                wsem.at[c % 2])
        for c in range(max(0, n_ch - 2), n_ch):
            wh[c].wait()

    return gath(idx2d, table)


def _comb_body(x2_ref, y0_ref, y1_ref, w1_ref, w2_ref, out_ref):
    out_ref[...] = (x2_ref[...] + w1_ref[...] * y0_ref[...]
                    + w2_ref[...] * y1_ref[...])


def _comb_call(x2, y01, w1, w2):
    grid = (_S // _BSQ,)
    nqb = _S // _BSQ
    return pl.pallas_call(
        _comb_body,
        grid=grid,
        in_specs=[
            pl.BlockSpec((_BSQ, _D), lambda i: (i, 0)),
            pl.BlockSpec((_BSQ, _D), lambda i: (i, 0)),
            pl.BlockSpec((_BSQ, _D), lambda i: (i + nqb, 0)),
            pl.BlockSpec((_BSQ, 1), lambda i: (i, 0)),
            pl.BlockSpec((_BSQ, 1), lambda i: (i, 0)),
        ],
        out_specs=pl.BlockSpec((_BSQ, _D), lambda i: (i, 0)),
        out_shape=jax.ShapeDtypeStruct((_T, _D), _F32),
        compiler_params=pltpu.CompilerParams(
            vmem_limit_bytes=100 * 1024 * 1024),
    )(x2, y01, y01, w1, w2)


def _dispatch(i1, i2):
    topi = jnp.concatenate([i1, i2], axis=1)
    flat_e = topi.reshape(_A)
    onehot = (flat_e[:, None] == jnp.arange(_E)[None, :]).astype(jnp.int32)
    rank = jnp.sum(onehot * (jnp.cumsum(onehot, axis=0) - onehot), axis=1)
    counts = jnp.sum(onehot, axis=0)
    nblk_e = (counts + _BT - 1) // _BT
    end_blk = jnp.cumsum(nblk_e)
    used = end_blk[-1]
    start_row = jnp.concatenate(
        [jnp.zeros((1,), jnp.int32), jnp.cumsum(nblk_e * _BT)[:-1]])
    pos = jnp.sum(onehot * start_row[None, :], axis=1) + rank
    tok = jnp.arange(_A, dtype=jnp.int32) // _K
    tok_sorted = (jnp.arange(_APAD, dtype=jnp.int32) % _T).at[pos].set(tok)
    bidx = jnp.arange(_NBLK, dtype=jnp.int32)
    eob = jnp.sum((bidx[:, None] >= end_blk[None, :]).astype(jnp.int32),
                  axis=1)
    is_real = (bidx < used).astype(jnp.int32)
    eob_last = jnp.sum((end_blk <= used - 1).astype(jnp.int32))
    eob = jnp.where(is_real > 0, eob, eob_last)
    meta = jnp.stack([eob, is_real]).astype(jnp.int32)
    return tok_sorted, meta, pos.reshape(_T, _K).astype(jnp.int32)


def kernel(hidden_states, ln1_w, Wq, Wk, Wv, q_norm_w, k_norm_w, Wo, ln2_w,
           router_W, W_gate, W_up, W_down):
    x = hidden_states.reshape(_T, _D)
    q2, k2, v2 = _qkv_call(x, ln1_w, Wq, Wk, Wv)
    q = _prep_call(q2, q_norm_w, _H)
    k = _prep_call(k2, k_norm_w, _KVH)
    v = v2.reshape(_S, _KVH, _HD).transpose(1, 0, 2).astype(_BF16)
    o = _attn_call(q, k, v)
    x2 = _proj_call(o.transpose(1, 0, 2).reshape(_S, _H * _HD), Wo, x)
    h2, w1, w2, i1, i2 = _router_call(x2, ln2_w, router_W)
    tok_sorted, meta, posr = _dispatch(i1, i2)
    x_sorted = _sc_gather(tok_sorted.reshape(_APAD // 24, 24), h2, _APAD, 24)
    y_sorted = _moe_call(meta, x_sorted, W_gate, W_up, W_down)
    pos01 = posr.transpose(1, 0).reshape(2 * _T // 16, 16)
    y01 = _sc_gather(pos01, y_sorted, 2 * _T, 16)
    out = _comb_call(x2, y01, w1, w2)
    return out.reshape(_B, _S, _D)


# speedup vs baseline: 2.5081x; 1.0162x over previous
import functools
import math

import jax
import jax.numpy as jnp
from jax.experimental import pallas as pl
from jax.experimental.pallas import tpu as pltpu
from jax.experimental.pallas import tpu_sc as plsc

_B, _S, _D = 1, 2048, 2048
_H, _KVH, _HD = 16, 4, 128
_E, _K, _F = 8, 2, 768
_EPS = 1e-6
_THETA = 10000.0
_T = _B * _S
_A = _T * _K
_BT = 256
_NBLK = _A // _BT + _E
_APAD = _NBLK * _BT
_BQ = 512
_BSA = 256
_SCALE = 1.0 / math.sqrt(_HD)
_F32 = jnp.float32
_BF16 = jnp.bfloat16
_BSQ = 512


def _rms(x, w):
    return x * jax.lax.rsqrt(jnp.mean(x * x, axis=-1, keepdims=True) + _EPS) * w


def _qkv_body(x_ref, ln1_ref, wq_ref, wk_ref, wv_ref, q_ref, k_ref, v_ref):
    x = x_ref[...]
    h = _rms(x, ln1_ref[...]).astype(_BF16)
    q_ref[...] = jnp.dot(h, wq_ref[...].astype(_BF16),
                         preferred_element_type=_F32)
    k_ref[...] = jnp.dot(h, wk_ref[...].astype(_BF16),
                         preferred_element_type=_F32)
    v_ref[...] = jnp.dot(h, wv_ref[...].astype(_BF16),
                         preferred_element_type=_F32)


def _qkv_call(x, ln1_w, wq, wk, wv):
    grid = (_S // _BSQ,)
    return pl.pallas_call(
        _qkv_body,
        grid=grid,
        in_specs=[
            pl.BlockSpec((_BSQ, _D), lambda i: (i, 0)),
            pl.BlockSpec((1, _D), lambda i: (0, 0)),
            pl.BlockSpec((_D, _H * _HD), lambda i: (0, 0)),
            pl.BlockSpec((_D, _KVH * _HD), lambda i: (0, 0)),
            pl.BlockSpec((_D, _KVH * _HD), lambda i: (0, 0)),
        ],
        out_specs=[
            pl.BlockSpec((_BSQ, _H * _HD), lambda i: (i, 0)),
            pl.BlockSpec((_BSQ, _KVH * _HD), lambda i: (i, 0)),
            pl.BlockSpec((_BSQ, _KVH * _HD), lambda i: (i, 0)),
        ],
        out_shape=[
            jax.ShapeDtypeStruct((_S, _H * _HD), _F32),
            jax.ShapeDtypeStruct((_S, _KVH * _HD), _F32),
            jax.ShapeDtypeStruct((_S, _KVH * _HD), _F32),
        ],
        compiler_params=pltpu.CompilerParams(
            vmem_limit_bytes=100 * 1024 * 1024),
    )(x, ln1_w.reshape(1, _D), wq, wk, wv)


def _prep_body(x_ref, nw_ref, o_ref):
    half = _HD // 2
    x = _rms(x_ref[...], nw_ref[...])
    pos = jax.lax.broadcasted_iota(jnp.int32, (_S, 1), 0).astype(_F32)
    inv = jnp.exp(jax.lax.broadcasted_iota(jnp.int32, (1, half), 1)
                  .astype(_F32) * (-math.log(_THETA) / half))
    f = pos * inv
    cos = jnp.cos(f)
    sin = jnp.sin(f)
    x1 = x[:, :half]
    x2 = x[:, half:]
    o_ref[0] = jnp.concatenate([x1 * cos - x2 * sin, x2 * cos + x1 * sin],
                               axis=-1).astype(_BF16)


def _prep_call(x2d, nw, nh):
    return pl.pallas_call(
        _prep_body,
        grid=(nh,),
        in_specs=[
            pl.BlockSpec((_S, _HD), lambda h: (0, h)),
            pl.BlockSpec((1, _HD), lambda h: (0, 0)),
        ],
        out_specs=pl.BlockSpec((1, _S, _HD), lambda h: (h, 0, 0)),
        out_shape=jax.ShapeDtypeStruct((nh, _S, _HD), _BF16),
        compiler_params=pltpu.CompilerParams(
            vmem_limit_bytes=100 * 1024 * 1024),
    )(x2d, nw.reshape(1, _HD))


def _attn_body(q_ref, k_ref, v_ref, o_ref):
    qb = pl.program_id(1)
    q = q_ref[0]

    def step(j, carry, masked):
        m, l, acc = carry
        k = k_ref[0, pl.ds(j * _BQ, _BQ), :]
        s = jax.lax.dot_general(q, k, (((1,), (1,)), ((), ())),
                                preferred_element_type=_F32) * _SCALE
        if masked:
            row = jax.lax.broadcasted_iota(jnp.int32, (_BQ, _BQ), 0)
            col = jax.lax.broadcasted_iota(jnp.int32, (_BQ, _BQ), 1)
            s = jnp.where(row >= col, s, -1e30)
        mj = jnp.max(s, axis=-1, keepdims=True)
        mn = jnp.maximum(m, mj)
        p = jnp.exp(s - mn)
        c = jnp.exp(m - mn)
        v = v_ref[0, pl.ds(j * _BQ, _BQ), :]
        acc = acc * c + jnp.dot(p.astype(_BF16), v,
                                preferred_element_type=_F32)
        l = l * c + jnp.sum(p, axis=-1, keepdims=True)
        return mn, l, acc

    init = (jnp.full((_BQ, 1), -1e30, _F32),
            jnp.zeros((_BQ, 1), _F32),
            jnp.zeros((_BQ, _HD), _F32))
    m, l, acc = jax.lax.fori_loop(
        0, qb, lambda j, cr: step(j, cr, False), init)
    m, l, acc = step(qb, (m, l, acc), True)
    o_ref[0] = (acc / l).astype(_BF16)


def _attn_call(q, k, v):
    rep = _H // _KVH
    grid = (_H, _S // _BQ)
    return pl.pallas_call(
        _attn_body,
        grid=grid,
        in_specs=[
            pl.BlockSpec((1, _BQ, _HD), lambda h, qb: (h, qb, 0)),
            pl.BlockSpec((1, _S, _HD), lambda h, qb: (h // rep, 0, 0)),
            pl.BlockSpec((1, _S, _HD), lambda h, qb: (h // rep, 0, 0)),
        ],
        out_specs=pl.BlockSpec((1, _BQ, _HD), lambda h, qb: (h, qb, 0)),
        out_shape=jax.ShapeDtypeStruct((_H, _S, _HD), _BF16),
        compiler_params=pltpu.CompilerParams(
            vmem_limit_bytes=100 * 1024 * 1024),
    )(q, k, v)


def _router_body(o_ref, wo_ref, res_ref, ln2_ref, rw_ref, x2_ref, h_ref,
                 w1_ref, w2_ref, i1_ref, i2_ref):
    x = res_ref[...] + jnp.dot(o_ref[...], wo_ref[...].astype(_BF16),
                               preferred_element_type=_F32)
    x2_ref[...] = x
    h = _rms(x, ln2_ref[...])
    h_ref[...] = h
    logits = jnp.dot(h, rw_ref[...], preferred_element_type=_F32)
    m = jnp.max(logits, axis=-1, keepdims=True)
    p = jnp.exp(logits - m)
    p = p / jnp.sum(p, axis=-1, keepdims=True)
    ids = jax.lax.broadcasted_iota(jnp.int32, (_BSQ, _E), 1)
    m1 = jnp.max(p, axis=-1, keepdims=True)
    i1 = jnp.min(jnp.where(p == m1, ids, _E), axis=-1, keepdims=True)
    pm = jnp.where(ids == i1, -1.0, p)
    m2 = jnp.max(pm, axis=-1, keepdims=True)
    i2 = jnp.min(jnp.where(pm == m2, ids, _E), axis=-1, keepdims=True)
    sw = m1 + m2
    w1_ref[...] = m1 / sw
    w2_ref[...] = m2 / sw
    i1_ref[...] = i1
    i2_ref[...] = i2


def _router_call(o, wo, res, ln2_w, rw):
    grid = (_S // _BSQ,)
    return pl.pallas_call(
        _router_body,
        grid=grid,
        in_specs=[
            pl.BlockSpec((_BSQ, _H * _HD), lambda i: (i, 0)),
            pl.BlockSpec((_H * _HD, _D), lambda i: (0, 0)),
            pl.BlockSpec((_BSQ, _D), lambda i: (i, 0)),
            pl.BlockSpec((1, _D), lambda i: (0, 0)),
            pl.BlockSpec((_D, _E), lambda i: (0, 0)),
        ],
        out_specs=[
            pl.BlockSpec((_BSQ, _D), lambda i: (i, 0)),
            pl.BlockSpec((_BSQ, _D), lambda i: (i, 0)),
            pl.BlockSpec((_BSQ, 1), lambda i: (i, 0)),
            pl.BlockSpec((_BSQ, 1), lambda i: (i, 0)),
            pl.BlockSpec((_BSQ, 1), lambda i: (i, 0)),
            pl.BlockSpec((_BSQ, 1), lambda i: (i, 0)),
        ],
        out_shape=[
            jax.ShapeDtypeStruct((_T, _D), _F32),
            jax.ShapeDtypeStruct((_T, _D), _F32),
            jax.ShapeDtypeStruct((_T, 1), _F32),
            jax.ShapeDtypeStruct((_T, 1), _F32),
            jax.ShapeDtypeStruct((_T, 1), jnp.int32),
            jax.ShapeDtypeStruct((_T, 1), jnp.int32),
        ],
        compiler_params=pltpu.CompilerParams(
            vmem_limit_bytes=100 * 1024 * 1024),
    )(o, wo, res, ln2_w.reshape(1, _D), rw)


def _moe_body(meta_ref, x_ref, wg_ref, wu_ref, wd_ref, y_ref,
              wg16_ref, wu16_ref, wd16_ref):
    b = pl.program_id(0)
    live = meta_ref[1, b] > 0
    changed = jnp.logical_or(
        b == 0, meta_ref[0, b] != meta_ref[0, jnp.maximum(b - 1, 0)])

    @pl.when(jnp.logical_and(live, changed))
    def _():
        wg16_ref[...] = wg_ref[0].astype(_BF16)
        wu16_ref[...] = wu_ref[0].astype(_BF16)
        wd16_ref[...] = wd_ref[0].astype(_BF16)

    @pl.when(live)
    def _():
        x = x_ref[...].astype(_BF16)
        g = jnp.dot(x, wg16_ref[...], preferred_element_type=_F32)
        u = jnp.dot(x, wu16_ref[...], preferred_element_type=_F32)
        a = (g * jax.lax.logistic(g) * u).astype(_BF16)
        y_ref[...] = jnp.dot(a, wd16_ref[...], preferred_element_type=_F32)

    @pl.when(jnp.logical_not(live))
    def _():
        y_ref[...] = jnp.zeros_like(y_ref)


def _moe_call(meta, x_sorted, wg, wu, wd):
    grid_spec = pltpu.PrefetchScalarGridSpec(
        num_scalar_prefetch=1,
        grid=(_NBLK,),
        in_specs=[
            pl.BlockSpec((_BT, _D), lambda b, m: (b, 0)),
            pl.BlockSpec((1, _D, _F), lambda b, m: (m[0, b], 0, 0)),
            pl.BlockSpec((1, _D, _F), lambda b, m: (m[0, b], 0, 0)),
            pl.BlockSpec((1, _F, _D), lambda b, m: (m[0, b], 0, 0)),
        ],
        out_specs=pl.BlockSpec((_BT, _D), lambda b, m: (b, 0)),
        scratch_shapes=[
            pltpu.VMEM((_D, _F), _BF16),
            pltpu.VMEM((_D, _F), _BF16),
            pltpu.VMEM((_F, _D), _BF16),
        ],
    )
    return pl.pallas_call(
        _moe_body,
        grid_spec=grid_spec,
        out_shape=jax.ShapeDtypeStruct((_APAD, _D), _F32),
        compiler_params=pltpu.CompilerParams(
            dimension_semantics=("arbitrary",),
            vmem_limit_bytes=110 * 1024 * 1024),
    )(meta, x_sorted, wg, wu, wd)


_NC, _NS = 2, 16
_NW = _NC * _NS
_DW = _D


def _sc_gather(idx2d, table, n_rows, chunk):
    per_w = n_rows // _NW
    n_ch = per_w // chunk
    mesh = plsc.VectorSubcoreMesh(core_axis_name="c", subcore_axis_name="s")

    @functools.partial(
        pl.kernel, mesh=mesh,
        out_type=jax.ShapeDtypeStruct((n_rows, _DW), _F32),
        scratch_types=[
            pltpu.VMEM((n_ch, chunk), jnp.int32),
            pltpu.VMEM((2, chunk, _DW), _F32),
            pltpu.SemaphoreType.DMA((2,)),
            pltpu.SemaphoreType.DMA((2,)),
        ],
    )
    def gath(idx_hbm, tab_hbm, out_hbm, idx_v, buf_v, gsem, wsem):
        wid = jax.lax.axis_index("s") * _NC + jax.lax.axis_index("c")
        base = wid * per_w
        pltpu.sync_copy(idx_hbm.at[pl.ds(wid * n_ch, n_ch)], idx_v)
        gh = {}
        wh = {}

        def fire(c):
            gh[c] = pltpu.async_copy(tab_hbm.at[idx_v.at[c]],
                                     buf_v.at[c % 2], gsem.at[c % 2])

        fire(0)
        for c in range(n_ch):
            if c + 1 < n_ch:
                if c - 1 >= 0:
                    wh[c - 1].wait()
                fire(c + 1)
            gh[c].wait()
            wh[c] = pltpu.async_copy(
                buf_v.at[c % 2],
                out_hbm.at[pl.ds(base + c * chunk, chunk)],
                wsem.at[c % 2])
        for c in range(max(0, n_ch - 2), n_ch):
            wh[c].wait()

    return gath(idx2d, table)


def _comb_body(x2_ref, y0_ref, y1_ref, w1_ref, w2_ref, out_ref):
    out_ref[...] = (x2_ref[...] + w1_ref[...] * y0_ref[...]
                    + w2_ref[...] * y1_ref[...])


def _comb_call(x2, y01, w1, w2):
    grid = (_S // _BSQ,)
    nqb = _S // _BSQ
    return pl.pallas_call(
        _comb_body,
        grid=grid,
        in_specs=[
            pl.BlockSpec((_BSQ, _D), lambda i: (i, 0)),
            pl.BlockSpec((_BSQ, _D), lambda i: (i, 0)),
            pl.BlockSpec((_BSQ, _D), lambda i: (i + nqb, 0)),
            pl.BlockSpec((_BSQ, 1), lambda i: (i, 0)),
            pl.BlockSpec((_BSQ, 1), lambda i: (i, 0)),
        ],
        out_specs=pl.BlockSpec((_BSQ, _D), lambda i: (i, 0)),
        out_shape=jax.ShapeDtypeStruct((_T, _D), _F32),
        compiler_params=pltpu.CompilerParams(
            vmem_limit_bytes=100 * 1024 * 1024),
    )(x2, y01, y01, w1, w2)


def _dispatch(i1, i2):
    topi = jnp.concatenate([i1, i2], axis=1)
    flat_e = topi.reshape(_A)
    onehot = (flat_e[:, None] == jnp.arange(_E)[None, :]).astype(jnp.int32)
    rank = jnp.sum(onehot * (jnp.cumsum(onehot, axis=0) - onehot), axis=1)
    counts = jnp.sum(onehot, axis=0)
    nblk_e = (counts + _BT - 1) // _BT
    end_blk = jnp.cumsum(nblk_e)
    used = end_blk[-1]
    start_row = jnp.concatenate(
        [jnp.zeros((1,), jnp.int32), jnp.cumsum(nblk_e * _BT)[:-1]])
    pos = jnp.sum(onehot * start_row[None, :], axis=1) + rank
    tok = jnp.arange(_A, dtype=jnp.int32) // _K
    tok_sorted = (jnp.arange(_APAD, dtype=jnp.int32) % _T).at[pos].set(tok)
    bidx = jnp.arange(_NBLK, dtype=jnp.int32)
    eob = jnp.sum((bidx[:, None] >= end_blk[None, :]).astype(jnp.int32),
                  axis=1)
    is_real = (bidx < used).astype(jnp.int32)
    eob_last = jnp.sum((end_blk <= used - 1).astype(jnp.int32))
    eob = jnp.where(is_real > 0, eob, eob_last)
    meta = jnp.stack([eob, is_real]).astype(jnp.int32)
    return tok_sorted, meta, pos.reshape(_T, _K).astype(jnp.int32)


def kernel(hidden_states, ln1_w, Wq, Wk, Wv, q_norm_w, k_norm_w, Wo, ln2_w,
           router_W, W_gate, W_up, W_down):
    x = hidden_states.reshape(_T, _D)
    q2, k2, v2 = _qkv_call(x, ln1_w, Wq, Wk, Wv)
    q = _prep_call(q2, q_norm_w, _H)
    k = _prep_call(k2, k_norm_w, _KVH)
    v = v2.reshape(_S, _KVH, _HD).transpose(1, 0, 2).astype(_BF16)
    o = _attn_call(q, k, v)
    x2, h2, w1, w2, i1, i2 = _router_call(
        o.transpose(1, 0, 2).reshape(_S, _H * _HD), Wo, x, ln2_w, router_W)
    tok_sorted, meta, posr = _dispatch(i1, i2)
    x_sorted = _sc_gather(tok_sorted.reshape(_APAD // 24, 24), h2, _APAD, 24)
    y_sorted = _moe_call(meta, x_sorted, W_gate, W_up, W_down)
    pos01 = posr.transpose(1, 0).reshape(2 * _T // 16, 16)
    y01 = _sc_gather(pos01, y_sorted, 2 * _T, 16)
    out = _comb_call(x2, y01, w1, w2)
    return out.reshape(_B, _S, _D)
